# Initial kernel scaffold; baseline (speedup 1.0000x reference)
import jax
import jax.numpy as jnp
from jax.experimental import pallas as pl


def kernel(user_id, items, Hs, mask, item_seq, user_price_seq, item_price_seq, user_count, nft_count, item_seq_len, user_emb, item_emb, adj_rows, adj_cols, adj_vals):
    raise NotImplementedError("write your pallas kernel here")


# speedup vs baseline: 10.0964x; 10.0964x over previous
import functools

import numpy as np
import jax
import jax.numpy as jnp
from jax import lax
from jax.experimental import pallas as pl
from jax.experimental.pallas import tpu as pltpu
from jax.experimental.pallas import tpu_sc as plsc

NC = 2
NS = 16
LANES = 16
K = 256
KJ = K // 128
RCH = 448
LRCH = 224
HCH = 784
HIST_W = 16

_MAGIC = np.int32(0x5F3759DF)


def _mesh():
    return plsc.VectorSubcoreMesh(
        core_axis_name="c", subcore_axis_name="s", num_cores=NC, num_subcores=NS
    )


def _rsqrt16(x):
    i = lax.bitcast_convert_type(x, jnp.int32)
    y = lax.bitcast_convert_type(
        _MAGIC - lax.shift_right_logical(i, 1), jnp.float32
    )
    for _ in range(3):
        y = y * (1.5 - 0.5 * x * y * y)
    return y


def _fill1d(ref, n, value):
    v = jnp.full((LANES,), value, jnp.float32)

    def body(r, _):
        ref[pl.ds(r * LANES, LANES)] = v
        return 0

    lax.fori_loop(0, n // LANES, body, 0)


def _scale_rows(src, dst, sc_v, base, n_rows):
    def grp(g, _):
        sc16 = sc_v[pl.ds(base + g * LANES, LANES)]
        for m in range(LANES):
            r = g * LANES + m
            sc = sc16[m]
            dst[r, pl.ds(0, 16)] = src[r, pl.ds(0, 16)] * sc
            dst[r, pl.ds(16, 16)] = src[r, pl.ds(16, 16)] * sc
        return 0

    lax.fori_loop(0, n_rows // LANES, grp, 0)


def _adjust(idx_v, off):
    for j in range(KJ):
        def body(k, _):
            idx_v[j, pl.ds(k * LANES, LANES)] = (
                idx_v[j, pl.ds(k * LANES, LANES)] + off
            )
            return 0

        lax.fori_loop(0, 128 // LANES, body, 0)


def _make_pre(e_pad, pad_half, n_u, n_i, tile_rows):
    nr_half = e_pad // 128
    n_chunks = e_pad // (NS * K)
    mesh = _mesh()

    @functools.partial(
        pl.kernel,
        mesh=mesh,
        compiler_params=pltpu.CompilerParams(use_tc_tiling_on_sc=False),
        out_type=(
            jax.ShapeDtypeStruct((NC * pad_half, 32), jnp.float32),
            jax.ShapeDtypeStruct((NC * pad_half,), jnp.float32),
            jax.ShapeDtypeStruct((NC * pad_half,), jnp.float32),
        ),
        scratch_types=[
            pltpu.VMEM_SHARED((pad_half,), jnp.float32),
            pltpu.VMEM((K,), jnp.float32),
            pltpu.VMEM((KJ, 128), jnp.int32),
            pltpu.VMEM((HCH,), jnp.float32),
            pltpu.VMEM((tile_rows,), jnp.float32),
            pltpu.VMEM((tile_rows,), jnp.float32),
            pltpu.VMEM((tile_rows,), jnp.float32),
            pltpu.VMEM((RCH, 32), jnp.float32),
            pltpu.VMEM((RCH, 32), jnp.float32),
        ],
    )
    def pre(rows_hbm, emb_hbm, t0_hbm, ivd_hbm, sqd_hbm,
            hist_sh, ones_v, idx_v, hbuf, di_v, iv_v, sd_v, ebuf, tbuf):
        c = lax.axis_index("c")
        s = lax.axis_index("s")
        base_n = s * tile_rows
        out0 = c * pad_half + base_n

        _fill1d(ones_v, K, 1.0)
        _fill1d(hbuf, HCH, 0.0)
        for k in range(tile_rows // HCH):
            pltpu.sync_copy(hbuf, hist_sh.at[pl.ds(base_n + k * HCH, HCH)])
        plsc.subcore_barrier()

        off_r = jnp.where(c == 0, jnp.int32(0), jnp.int32(n_u))

        def edge_body(i, _):
            g = c * nr_half + s * (n_chunks * KJ) + i * KJ
            pltpu.sync_copy(rows_hbm.at[pl.ds(g, KJ)], idx_v)
            _adjust(idx_v, -off_r)
            for j in range(KJ):
                pltpu.sync_copy(
                    ones_v.at[pl.ds(j * 128, 128)],
                    hist_sh.at[idx_v.at[j]],
                    add=True,
                )
            return 0

        lax.fori_loop(0, n_chunks, edge_body, 0)
        plsc.subcore_barrier()

        def hist_chunk(k, _):
            pltpu.sync_copy(hist_sh.at[pl.ds(base_n + k * HCH, HCH)], hbuf)

            def grp(g, _):
                deg = hbuf[pl.ds(g * LANES, LANES)]
                x = deg + 1e-7
                y = _rsqrt16(x)
                b = k * HCH + g * LANES
                di_v[pl.ds(b, LANES)] = y
                iv_v[pl.ds(b, LANES)] = 1.0 / x
                sd_v[pl.ds(b, LANES)] = 0.25 * x * y
                return 0

            lax.fori_loop(0, HCH // LANES, grp, 0)
            return 0

        lax.fori_loop(0, tile_rows // HCH, hist_chunk, 0)
        pltpu.sync_copy(iv_v, ivd_hbm.at[pl.ds(out0, tile_rows)])
        pltpu.sync_copy(sd_v, sqd_hbm.at[pl.ds(out0, tile_rows)])

        def t0_chunk(k, _):
            pltpu.sync_copy(emb_hbm.at[pl.ds(out0 + k * RCH, RCH)], ebuf)
            _scale_rows(ebuf, tbuf, di_v, k * RCH, RCH)
            pltpu.sync_copy(tbuf, t0_hbm.at[pl.ds(out0 + k * RCH, RCH)])
            return 0

        lax.fori_loop(0, tile_rows // RCH, t0_chunk, 0)

    return pre


def _make_layer(e_pad, pad_half, n_u, tile_rows):
    nr_half = e_pad // 128
    n_chunks = e_pad // (NS * K)
    mesh = _mesh()

    @functools.partial(
        pl.kernel,
        mesh=mesh,
        compiler_params=pltpu.CompilerParams(use_tc_tiling_on_sc=False),
        out_type=jax.ShapeDtypeStruct((NC * pad_half, 32), jnp.float32),
        scratch_types=[
            pltpu.VMEM_SHARED((pad_half, 32), jnp.float32),
            pltpu.VMEM((K, 32), jnp.float32),
            pltpu.VMEM((KJ, 128), jnp.int32),
            pltpu.VMEM((KJ, 128), jnp.int32),
            pltpu.VMEM((LRCH, 32), jnp.float32),
            pltpu.VMEM((tile_rows,), jnp.float32),
            pltpu.SemaphoreType.DMA,
        ],
    )
    def layer(rows_hbm, cols_hbm, t_hbm, ivd_hbm, out_hbm,
              acc_sh, gbuf, cidx_v, ridx_v, dbuf, iv_v, sem):
        c = lax.axis_index("c")
        s = lax.axis_index("s")
        base_n = s * tile_rows
        out0 = c * pad_half + base_n

        zv = jnp.zeros((LANES,), jnp.float32)

        def zrow(r, _):
            dbuf[r, pl.ds(0, 16)] = zv
            dbuf[r, pl.ds(16, 16)] = zv
            return 0

        lax.fori_loop(0, LRCH, zrow, 0)
        for k in range(tile_rows // LRCH):
            pltpu.sync_copy(dbuf, acc_sh.at[pl.ds(base_n + k * LRCH, LRCH)])
        plsc.subcore_barrier()

        off_r = jnp.where(c == 0, jnp.int32(0), jnp.int32(n_u))
        off_c = jnp.where(c == 0, jnp.int32(pad_half - n_u), jnp.int32(0))

        def edge_body(i, _):
            g = c * nr_half + s * (n_chunks * KJ) + i * KJ
            pltpu.sync_copy(cols_hbm.at[pl.ds(g, KJ)], cidx_v)
            pltpu.sync_copy(rows_hbm.at[pl.ds(g, KJ)], ridx_v)
            _adjust(cidx_v, off_c)
            _adjust(ridx_v, -off_r)
            for j in range(KJ):
                pltpu.async_copy(
                    t_hbm.at[cidx_v.at[j]],
                    gbuf.at[pl.ds(j * 128, 128)],
                    sem,
                ).wait()
            for j in range(KJ):
                pltpu.sync_copy(
                    gbuf.at[pl.ds(j * 128, 128)],
                    acc_sh.at[ridx_v.at[j]],
                    add=True,
                )
            return 0

        lax.fori_loop(0, n_chunks, edge_body, 0)
        plsc.subcore_barrier()

        pltpu.sync_copy(ivd_hbm.at[pl.ds(out0, tile_rows)], iv_v)

        def drain(k, _):
            pltpu.sync_copy(acc_sh.at[pl.ds(base_n + k * LRCH, LRCH)], dbuf)
            _scale_rows(dbuf, dbuf, iv_v, k * LRCH, LRCH)
            pltpu.sync_copy(dbuf, out_hbm.at[pl.ds(out0 + k * LRCH, LRCH)])
            return 0

        lax.fori_loop(0, tile_rows // LRCH, drain, 0)

    return layer


def _make_final(pad_half, n_i, batch, tile_rows):
    upt = batch // NS
    uj = upt // 128
    mesh = _mesh()

    @functools.partial(
        pl.kernel,
        mesh=mesh,
        compiler_params=pltpu.CompilerParams(use_tc_tiling_on_sc=False),
        out_type=(
            jax.ShapeDtypeStruct((batch, 32), jnp.float32),
            jax.ShapeDtypeStruct((n_i, 32), jnp.float32),
        ),
        scratch_types=[
            pltpu.VMEM((RCH,), jnp.float32),
            pltpu.VMEM((RCH, 32), jnp.float32),
            pltpu.VMEM((RCH, 32), jnp.float32),
            pltpu.VMEM((RCH, 32), jnp.float32),
            pltpu.VMEM((RCH, 32), jnp.float32),
            pltpu.VMEM((uj, 128), jnp.int32),
            pltpu.VMEM((RCH, 32), jnp.float32),
            pltpu.SemaphoreType.DMA,
        ],
    )
    def final(uid_hbm, t0_hbm, t1_hbm, t2_hbm, t3_hbm, sqd_hbm,
              uout_hbm, iout_hbm,
              sdb, b0, b1, b2, b3, uid_v, obuf, sem):
        c = lax.axis_index("c")
        s = lax.axis_index("s")
        tts = (t0_hbm, t1_hbm, t2_hbm, t3_hbm)
        bbs = (b0, b1, b2, b3)

        @pl.when(c == 0)
        def _user():
            pltpu.sync_copy(uid_hbm.at[pl.ds(s * uj, uj)], uid_v)
            for j in range(uj):
                pltpu.async_copy(
                    sqd_hbm.at[uid_v.at[j]], sdb.at[pl.ds(j * 128, 128)], sem
                ).wait()
            for t, b in zip(tts, bbs):
                for j in range(uj):
                    pltpu.async_copy(
                        t.at[uid_v.at[j]], b.at[pl.ds(j * 128, 128)], sem
                    ).wait()

            for j in range(uj):
                def grp(g, _):
                    sc16 = sdb[pl.ds(j * 128 + g * LANES, LANES)]
                    for m in range(LANES):
                        r = j * 128 + g * LANES + m
                        sc = sc16[m]
                        obuf[r, pl.ds(0, 16)] = (
                            b0[r, pl.ds(0, 16)] + b1[r, pl.ds(0, 16)]
                            + b2[r, pl.ds(0, 16)] + b3[r, pl.ds(0, 16)]
                        ) * sc
                        obuf[r, pl.ds(16, 16)] = (
                            b0[r, pl.ds(16, 16)] + b1[r, pl.ds(16, 16)]
                            + b2[r, pl.ds(16, 16)] + b3[r, pl.ds(16, 16)]
                        ) * sc
                    return 0

                lax.fori_loop(0, 128 // LANES, grp, 0)
            pltpu.sync_copy(
                obuf.at[pl.ds(0, upt)], uout_hbm.at[pl.ds(s * upt, upt)]
            )

        @pl.when(c == 1)
        def _item():
            def chunk(k, _):
                row0 = jnp.minimum(s * tile_rows + k * RCH, n_i - RCH)
                pltpu.sync_copy(sqd_hbm.at[pl.ds(pad_half + row0, RCH)], sdb)
                for t, b in zip(tts, bbs):
                    pltpu.sync_copy(t.at[pl.ds(pad_half + row0, RCH)], b)

                def grp(g, _):
                    sc16 = sdb[pl.ds(g * LANES, LANES)]
                    for m in range(LANES):
                        r = g * LANES + m
                        sc = sc16[m]
                        obuf[r, pl.ds(0, 16)] = (
                            b0[r, pl.ds(0, 16)] + b1[r, pl.ds(0, 16)]
                            + b2[r, pl.ds(0, 16)] + b3[r, pl.ds(0, 16)]
                        ) * sc
                        obuf[r, pl.ds(16, 16)] = (
                            b0[r, pl.ds(16, 16)] + b1[r, pl.ds(16, 16)]
                            + b2[r, pl.ds(16, 16)] + b3[r, pl.ds(16, 16)]
                        ) * sc
                    return 0

                lax.fori_loop(0, RCH // LANES, grp, 0)
                pltpu.sync_copy(obuf, iout_hbm.at[pl.ds(row0, RCH)])
                return 0

            lax.fori_loop(0, -(-tile_rows // RCH), chunk, 0)

    return final


def kernel(user_id, items, Hs, mask, item_seq, user_price_seq, item_price_seq,
           user_count, nft_count, item_seq_len, user_emb, item_emb,
           adj_rows, adj_cols, adj_vals):
    n_u, d = user_emb.shape
    n_i = item_emb.shape[0]
    e = adj_rows.shape[0] // 2
    batch = user_id.shape[0]

    tile_rows = -(-max(n_u, n_i) // NS)
    tile_rows = -(-tile_rows // RCH) * RCH
    pad_half = NS * tile_rows

    ept = -(-e // (NS * K)) * K
    e_pad = ept * NS
    nr_half = e_pad // 128

    pad_e = e_pad - e
    rows_p = jnp.concatenate([
        jnp.pad(adj_rows[:e], (0, pad_e), constant_values=n_u + 8),
        jnp.pad(adj_rows[e:], (0, pad_e), constant_values=n_u + n_i + 8),
    ]).reshape(2 * nr_half, 128)
    cols_p = jnp.concatenate([
        jnp.pad(adj_cols[:e], (0, pad_e), constant_values=n_u),
        jnp.pad(adj_cols[e:], (0, pad_e), constant_values=0),
    ]).reshape(2 * nr_half, 128)
    emb_pad = jnp.concatenate([
        user_emb,
        jnp.zeros((pad_half - n_u, d), jnp.float32),
        item_emb,
        jnp.zeros((pad_half - n_i, d), jnp.float32),
    ])

    pre = _make_pre(e_pad, pad_half, n_u, n_i, tile_rows)
    layer = _make_layer(e_pad, pad_half, n_u, tile_rows)
    final = _make_final(pad_half, n_i, batch, tile_rows)

    t0, invdeg, sqd4 = pre(rows_p, emb_pad)
    t1 = layer(rows_p, cols_p, t0, invdeg)
    t2 = layer(rows_p, cols_p, t1, invdeg)
    t3 = layer(rows_p, cols_p, t2, invdeg)
    uid2d = user_id.reshape(batch // 128, 128)
    user_out, item_all = final(uid2d, t0, t1, t2, t3, sqd4)
    return (user_out, item_all)


# speedup vs baseline: 17.9012x; 1.7730x over previous
import functools

import numpy as np
import jax
import jax.numpy as jnp
from jax import lax
from jax.experimental import pallas as pl
from jax.experimental.pallas import tpu as pltpu
from jax.experimental.pallas import tpu_sc as plsc

NC = 2
NS = 16
LANES = 16
K = 256
KJ = K // 128
RCH = 448
LRCH = 224
HCH = 784
HIST_W = 16

_MAGIC = np.int32(0x5F3759DF)


def _mesh():
    return plsc.VectorSubcoreMesh(
        core_axis_name="c", subcore_axis_name="s", num_cores=NC, num_subcores=NS
    )


def _rsqrt16(x):
    i = lax.bitcast_convert_type(x, jnp.int32)
    y = lax.bitcast_convert_type(
        _MAGIC - lax.shift_right_logical(i, 1), jnp.float32
    )
    for _ in range(3):
        y = y * (1.5 - 0.5 * x * y * y)
    return y


def _fill1d(ref, n, value):
    v = jnp.full((LANES,), value, jnp.float32)

    def body(r, _):
        ref[pl.ds(r * LANES, LANES)] = v
        return 0

    lax.fori_loop(0, n // LANES, body, 0)


def _scale_rows(src, dst, sc_v, base, n_rows):
    def grp(g, _):
        sc16 = sc_v[pl.ds(base + g * LANES, LANES)]
        for m in range(LANES):
            r = g * LANES + m
            sc = sc16[m]
            dst[r, pl.ds(0, 16)] = src[r, pl.ds(0, 16)] * sc
            dst[r, pl.ds(16, 16)] = src[r, pl.ds(16, 16)] * sc
        return 0

    lax.fori_loop(0, n_rows // LANES, grp, 0)


def _adjust(idx_v, off):
    for j in range(KJ):
        def body(k, _):
            idx_v[j, pl.ds(k * LANES, LANES)] = (
                idx_v[j, pl.ds(k * LANES, LANES)] + off
            )
            return 0

        lax.fori_loop(0, 128 // LANES, body, 0)


def _make_pre(e_pad, pad_half, n_u, n_i, tile_rows):
    nr_half = e_pad // 128
    n_chunks = e_pad // (NS * K)
    mesh = _mesh()

    @functools.partial(
        pl.kernel,
        mesh=mesh,
        compiler_params=pltpu.CompilerParams(use_tc_tiling_on_sc=False),
        out_type=(
            jax.ShapeDtypeStruct((NC * pad_half, 32), jnp.float32),
            jax.ShapeDtypeStruct((NC * pad_half,), jnp.float32),
            jax.ShapeDtypeStruct((NC * pad_half,), jnp.float32),
        ),
        scratch_types=[
            pltpu.VMEM_SHARED((pad_half,), jnp.float32),
            pltpu.VMEM((K,), jnp.float32),
            pltpu.VMEM((KJ, 128), jnp.int32),
            pltpu.VMEM((HCH,), jnp.float32),
            pltpu.VMEM((tile_rows,), jnp.float32),
            pltpu.VMEM((tile_rows,), jnp.float32),
            pltpu.VMEM((tile_rows,), jnp.float32),
            pltpu.VMEM((RCH, 32), jnp.float32),
            pltpu.VMEM((RCH, 32), jnp.float32),
        ],
    )
    def pre(rows_hbm, emb_hbm, t0_hbm, ivd_hbm, sqd_hbm,
            hist_sh, ones_v, idx_v, hbuf, di_v, iv_v, sd_v, ebuf, tbuf):
        c = lax.axis_index("c")
        s = lax.axis_index("s")
        base_n = s * tile_rows
        out0 = c * pad_half + base_n

        _fill1d(ones_v, K, 1.0)
        _fill1d(hbuf, HCH, 0.0)
        for k in range(tile_rows // HCH):
            pltpu.sync_copy(hbuf, hist_sh.at[pl.ds(base_n + k * HCH, HCH)])
        plsc.subcore_barrier()

        off_r = jnp.where(c == 0, jnp.int32(0), jnp.int32(n_u))

        def edge_body(i, _):
            g = c * nr_half + s * (n_chunks * KJ) + i * KJ
            pltpu.sync_copy(rows_hbm.at[pl.ds(g, KJ)], idx_v)
            _adjust(idx_v, -off_r)
            for j in range(KJ):
                pltpu.sync_copy(
                    ones_v.at[pl.ds(j * 128, 128)],
                    hist_sh.at[idx_v.at[j]],
                    add=True,
                )
            return 0

        lax.fori_loop(0, n_chunks, edge_body, 0)
        plsc.subcore_barrier()

        def hist_chunk(k, _):
            pltpu.sync_copy(hist_sh.at[pl.ds(base_n + k * HCH, HCH)], hbuf)

            def grp(g, _):
                deg = hbuf[pl.ds(g * LANES, LANES)]
                x = deg + 1e-7
                y = _rsqrt16(x)
                b = k * HCH + g * LANES
                di_v[pl.ds(b, LANES)] = y
                iv_v[pl.ds(b, LANES)] = 1.0 / x
                sd_v[pl.ds(b, LANES)] = 0.25 * x * y
                return 0

            lax.fori_loop(0, HCH // LANES, grp, 0)
            return 0

        lax.fori_loop(0, tile_rows // HCH, hist_chunk, 0)
        pltpu.sync_copy(iv_v, ivd_hbm.at[pl.ds(out0, tile_rows)])
        pltpu.sync_copy(sd_v, sqd_hbm.at[pl.ds(out0, tile_rows)])

        def t0_chunk(k, _):
            pltpu.sync_copy(emb_hbm.at[pl.ds(out0 + k * RCH, RCH)], ebuf)
            _scale_rows(ebuf, tbuf, di_v, k * RCH, RCH)
            pltpu.sync_copy(tbuf, t0_hbm.at[pl.ds(out0 + k * RCH, RCH)])
            return 0

        lax.fori_loop(0, tile_rows // RCH, t0_chunk, 0)

    return pre


def _make_layer(e_pad, pad_half, n_u, tile_rows):
    nr_half = e_pad // 128
    n_chunks = e_pad // (NS * K)
    mesh = _mesh()

    @functools.partial(
        pl.kernel,
        mesh=mesh,
        compiler_params=pltpu.CompilerParams(use_tc_tiling_on_sc=False),
        out_type=jax.ShapeDtypeStruct((NC * pad_half, 32), jnp.float32),
        scratch_types=[
            pltpu.VMEM_SHARED((pad_half, 32), jnp.float32),
            pltpu.VMEM((2 * K, 32), jnp.float32),
            pltpu.VMEM((2 * KJ, 128), jnp.int32),
            pltpu.VMEM((2 * KJ, 128), jnp.int32),
            pltpu.VMEM((tile_rows,), jnp.float32),
            pltpu.SemaphoreType.DMA,
            pltpu.SemaphoreType.DMA,
        ],
    )
    def layer(rows_hbm, cols_hbm, t_hbm, ivd_hbm, out_hbm,
              acc_sh, gbuf, cidx_v, ridx_v, iv_v, sem_g, sem_s):
        c = lax.axis_index("c")
        s = lax.axis_index("s")
        base_n = s * tile_rows
        out0 = c * pad_half + base_n

        zv = jnp.zeros((LANES,), jnp.float32)

        def zrow(r, _):
            gbuf[r, pl.ds(0, 16)] = zv
            gbuf[r, pl.ds(16, 16)] = zv
            return 0

        lax.fori_loop(0, LRCH, zrow, 0)
        for k in range(tile_rows // LRCH):
            pltpu.sync_copy(
                gbuf.at[pl.ds(0, LRCH)],
                acc_sh.at[pl.ds(base_n + k * LRCH, LRCH)],
            )
        plsc.subcore_barrier()

        off_r = jnp.where(c == 0, jnp.int32(0), jnp.int32(n_u))
        off_c = jnp.where(c == 0, jnp.int32(pad_half - n_u), jnp.int32(0))
        gbase = c * nr_half + s * (n_chunks * KJ)

        def load_and_gather(i, slot):
            g = gbase + i * KJ
            pltpu.sync_copy(cols_hbm.at[pl.ds(g, KJ)],
                            cidx_v.at[pl.ds(slot * KJ, KJ)])
            pltpu.sync_copy(rows_hbm.at[pl.ds(g, KJ)],
                            ridx_v.at[pl.ds(slot * KJ, KJ)])
            for j in range(KJ):
                def adj(k, _):
                    cidx_v[slot * KJ + j, pl.ds(k * LANES, LANES)] = (
                        cidx_v[slot * KJ + j, pl.ds(k * LANES, LANES)] + off_c
                    )
                    ridx_v[slot * KJ + j, pl.ds(k * LANES, LANES)] = (
                        ridx_v[slot * KJ + j, pl.ds(k * LANES, LANES)] - off_r
                    )
                    return 0

                lax.fori_loop(0, 128 // LANES, adj, 0)
            for j in range(KJ):
                pltpu.async_copy(
                    t_hbm.at[cidx_v.at[slot * KJ + j]],
                    gbuf.at[pl.ds(slot * K + j * 128, 128)],
                    sem_g,
                )

        def wait_gather(slot):
            for j in range(KJ):
                pltpu.make_async_copy(
                    t_hbm.at[cidx_v.at[slot * KJ + j]],
                    gbuf.at[pl.ds(slot * K + j * 128, 128)],
                    sem_g,
                ).wait()

        def start_scatter(slot):
            for j in range(KJ):
                pltpu.async_copy(
                    gbuf.at[pl.ds(slot * K + j * 128, 128)],
                    acc_sh.at[ridx_v.at[slot * KJ + j]],
                    sem_s,
                    add=True,
                )

        def wait_scatter(slot):
            for j in range(KJ):
                pltpu.make_async_copy(
                    gbuf.at[pl.ds(slot * K + j * 128, 128)],
                    acc_sh.at[ridx_v.at[slot * KJ + j]],
                    sem_s,
                ).wait()

        load_and_gather(0, jnp.int32(0))

        def edge_body(i, _):
            slot = lax.rem(i, 2)
            nxt = 1 - slot

            @pl.when(i + 1 < n_chunks)
            def _prefetch():
                @pl.when(i >= 1)
                def _():
                    wait_scatter(nxt)

                load_and_gather(i + 1, nxt)

            wait_gather(slot)
            start_scatter(slot)
            return 0

        lax.fori_loop(0, n_chunks, edge_body, 0)
        wait_scatter(jnp.int32(n_chunks % 2))
        wait_scatter(jnp.int32((n_chunks - 1) % 2))
        plsc.subcore_barrier()

        pltpu.sync_copy(ivd_hbm.at[pl.ds(out0, tile_rows)], iv_v)

        def drain(k, _):
            pltpu.sync_copy(acc_sh.at[pl.ds(base_n + k * LRCH, LRCH)],
                            gbuf.at[pl.ds(0, LRCH)])
            _scale_rows(gbuf, gbuf, iv_v, k * LRCH, LRCH)
            pltpu.sync_copy(gbuf.at[pl.ds(0, LRCH)],
                            out_hbm.at[pl.ds(out0 + k * LRCH, LRCH)])
            return 0

        lax.fori_loop(0, tile_rows // LRCH, drain, 0)

    return layer


def _make_final(pad_half, n_i, batch, tile_rows):
    upt = batch // NS
    uj = upt // 128
    mesh = _mesh()

    @functools.partial(
        pl.kernel,
        mesh=mesh,
        compiler_params=pltpu.CompilerParams(use_tc_tiling_on_sc=False),
        out_type=(
            jax.ShapeDtypeStruct((batch, 32), jnp.float32),
            jax.ShapeDtypeStruct((n_i, 32), jnp.float32),
        ),
        scratch_types=[
            pltpu.VMEM((RCH,), jnp.float32),
            pltpu.VMEM((RCH, 32), jnp.float32),
            pltpu.VMEM((RCH, 32), jnp.float32),
            pltpu.VMEM((RCH, 32), jnp.float32),
            pltpu.VMEM((RCH, 32), jnp.float32),
            pltpu.VMEM((uj, 128), jnp.int32),
            pltpu.VMEM((RCH, 32), jnp.float32),
            pltpu.SemaphoreType.DMA,
        ],
    )
    def final(uid_hbm, t0_hbm, t1_hbm, t2_hbm, t3_hbm, sqd_hbm,
              uout_hbm, iout_hbm,
              sdb, b0, b1, b2, b3, uid_v, obuf, sem):
        c = lax.axis_index("c")
        s = lax.axis_index("s")
        tts = (t0_hbm, t1_hbm, t2_hbm, t3_hbm)
        bbs = (b0, b1, b2, b3)

        @pl.when(c == 0)
        def _user():
            pltpu.sync_copy(uid_hbm.at[pl.ds(s * uj, uj)], uid_v)
            for j in range(uj):
                pltpu.async_copy(
                    sqd_hbm.at[uid_v.at[j]], sdb.at[pl.ds(j * 128, 128)], sem
                ).wait()
            for t, b in zip(tts, bbs):
                for j in range(uj):
                    pltpu.async_copy(
                        t.at[uid_v.at[j]], b.at[pl.ds(j * 128, 128)], sem
                    ).wait()

            for j in range(uj):
                def grp(g, _):
                    sc16 = sdb[pl.ds(j * 128 + g * LANES, LANES)]
                    for m in range(LANES):
                        r = j * 128 + g * LANES + m
                        sc = sc16[m]
                        obuf[r, pl.ds(0, 16)] = (
                            b0[r, pl.ds(0, 16)] + b1[r, pl.ds(0, 16)]
                            + b2[r, pl.ds(0, 16)] + b3[r, pl.ds(0, 16)]
                        ) * sc
                        obuf[r, pl.ds(16, 16)] = (
                            b0[r, pl.ds(16, 16)] + b1[r, pl.ds(16, 16)]
                            + b2[r, pl.ds(16, 16)] + b3[r, pl.ds(16, 16)]
                        ) * sc
                    return 0

                lax.fori_loop(0, 128 // LANES, grp, 0)
            pltpu.sync_copy(
                obuf.at[pl.ds(0, upt)], uout_hbm.at[pl.ds(s * upt, upt)]
            )

        @pl.when(c == 1)
        def _item():
            def chunk(k, _):
                row0 = jnp.minimum(s * tile_rows + k * RCH, n_i - RCH)
                pltpu.sync_copy(sqd_hbm.at[pl.ds(pad_half + row0, RCH)], sdb)
                for t, b in zip(tts, bbs):
                    pltpu.sync_copy(t.at[pl.ds(pad_half + row0, RCH)], b)

                def grp(g, _):
                    sc16 = sdb[pl.ds(g * LANES, LANES)]
                    for m in range(LANES):
                        r = g * LANES + m
                        sc = sc16[m]
                        obuf[r, pl.ds(0, 16)] = (
                            b0[r, pl.ds(0, 16)] + b1[r, pl.ds(0, 16)]
                            + b2[r, pl.ds(0, 16)] + b3[r, pl.ds(0, 16)]
                        ) * sc
                        obuf[r, pl.ds(16, 16)] = (
                            b0[r, pl.ds(16, 16)] + b1[r, pl.ds(16, 16)]
                            + b2[r, pl.ds(16, 16)] + b3[r, pl.ds(16, 16)]
                        ) * sc
                    return 0

                lax.fori_loop(0, RCH // LANES, grp, 0)
                pltpu.sync_copy(obuf, iout_hbm.at[pl.ds(row0, RCH)])
                return 0

            lax.fori_loop(0, -(-tile_rows // RCH), chunk, 0)

    return final


def kernel(user_id, items, Hs, mask, item_seq, user_price_seq, item_price_seq,
           user_count, nft_count, item_seq_len, user_emb, item_emb,
           adj_rows, adj_cols, adj_vals):
    n_u, d = user_emb.shape
    n_i = item_emb.shape[0]
    e = adj_rows.shape[0] // 2
    batch = user_id.shape[0]

    tile_rows = -(-max(n_u, n_i) // NS)
    tile_rows = -(-tile_rows // RCH) * RCH
    pad_half = NS * tile_rows

    ept = -(-e // (NS * K)) * K
    e_pad = ept * NS
    nr_half = e_pad // 128

    pad_e = e_pad - e
    rows_p = jnp.concatenate([
        jnp.pad(adj_rows[:e], (0, pad_e), constant_values=n_u + 8),
        jnp.pad(adj_rows[e:], (0, pad_e), constant_values=n_u + n_i + 8),
    ]).reshape(2 * nr_half, 128)
    cols_p = jnp.concatenate([
        jnp.pad(adj_cols[:e], (0, pad_e), constant_values=n_u),
        jnp.pad(adj_cols[e:], (0, pad_e), constant_values=0),
    ]).reshape(2 * nr_half, 128)
    emb_pad = jnp.concatenate([
        user_emb,
        jnp.zeros((pad_half - n_u, d), jnp.float32),
        item_emb,
        jnp.zeros((pad_half - n_i, d), jnp.float32),
    ])

    pre = _make_pre(e_pad, pad_half, n_u, n_i, tile_rows)
    layer = _make_layer(e_pad, pad_half, n_u, tile_rows)
    final = _make_final(pad_half, n_i, batch, tile_rows)

    t0, invdeg, sqd4 = pre(rows_p, emb_pad)
    t1 = layer(rows_p, cols_p, t0, invdeg)
    t2 = layer(rows_p, cols_p, t1, invdeg)
    t3 = layer(rows_p, cols_p, t2, invdeg)
    uid2d = user_id.reshape(batch // 128, 128)
    user_out, item_all = final(uid2d, t0, t1, t2, t3, sqd4)
    return (user_out, item_all)


# speedup vs baseline: 26.7953x; 1.4968x over previous
import functools

import numpy as np
import jax
import jax.numpy as jnp
from jax import lax
from jax.experimental import pallas as pl
from jax.experimental.pallas import tpu as pltpu
from jax.experimental.pallas import tpu_sc as plsc

NC = 2
NS = 16
LANES = 16
K = 256
KJ = K // 128
RCH = 448
LRCH = 224
HCH = 784
HIST_W = 16

_MAGIC = np.int32(0x5F3759DF)


def _mesh():
    return plsc.VectorSubcoreMesh(
        core_axis_name="c", subcore_axis_name="s", num_cores=NC, num_subcores=NS
    )


def _rsqrt16(x):
    i = lax.bitcast_convert_type(x, jnp.int32)
    y = lax.bitcast_convert_type(
        _MAGIC - lax.shift_right_logical(i, 1), jnp.float32
    )
    for _ in range(3):
        y = y * (1.5 - 0.5 * x * y * y)
    return y


def _fill1d(ref, n, value):
    v = jnp.full((LANES,), value, jnp.float32)

    def body(r, _):
        ref[pl.ds(r * LANES, LANES)] = v
        return 0

    lax.fori_loop(0, n // LANES, body, 0)


def _scale_rows(src, dst, sc_v, base, n_rows):
    def grp(g, _):
        sc16 = sc_v[pl.ds(base + g * LANES, LANES)]
        for m in range(LANES):
            r = g * LANES + m
            sc = sc16[m]
            dst[r, pl.ds(0, 16)] = src[r, pl.ds(0, 16)] * sc
            dst[r, pl.ds(16, 16)] = src[r, pl.ds(16, 16)] * sc
        return 0

    lax.fori_loop(0, n_rows // LANES, grp, 0)


def _adjust(idx_v, off):
    for j in range(KJ):
        def body(k, _):
            idx_v[j, pl.ds(k * LANES, LANES)] = (
                idx_v[j, pl.ds(k * LANES, LANES)] + off
            )
            return 0

        lax.fori_loop(0, 128 // LANES, body, 0)


def _make_pre(e_pad, pad_half, n_u, n_i, tile_rows):
    nr_half = e_pad // 128
    n_chunks = e_pad // (NS * K)
    mesh = _mesh()

    @functools.partial(
        pl.kernel,
        mesh=mesh,
        compiler_params=pltpu.CompilerParams(use_tc_tiling_on_sc=False),
        out_type=(
            jax.ShapeDtypeStruct((NC * pad_half, 32), jnp.float32),
            jax.ShapeDtypeStruct((NC * pad_half,), jnp.float32),
            jax.ShapeDtypeStruct((NC * pad_half,), jnp.float32),
        ),
        scratch_types=[
            pltpu.VMEM_SHARED((pad_half,), jnp.float32),
            pltpu.VMEM((K,), jnp.float32),
            pltpu.VMEM((KJ, 128), jnp.int32),
            pltpu.VMEM((HCH,), jnp.float32),
            pltpu.VMEM((tile_rows,), jnp.float32),
            pltpu.VMEM((tile_rows,), jnp.float32),
            pltpu.VMEM((tile_rows,), jnp.float32),
            pltpu.VMEM((RCH, 32), jnp.float32),
            pltpu.VMEM((RCH, 32), jnp.float32),
        ],
    )
    def pre(rows_hbm, emb_hbm, t0_hbm, ivd_hbm, sqd_hbm,
            hist_sh, ones_v, idx_v, hbuf, di_v, iv_v, sd_v, ebuf, tbuf):
        c = lax.axis_index("c")
        s = lax.axis_index("s")
        base_n = s * tile_rows
        out0 = c * pad_half + base_n

        _fill1d(ones_v, K, 1.0)
        _fill1d(hbuf, HCH, 0.0)
        for k in range(tile_rows // HCH):
            pltpu.sync_copy(hbuf, hist_sh.at[pl.ds(base_n + k * HCH, HCH)])
        plsc.subcore_barrier()

        off_r = jnp.where(c == 0, jnp.int32(0), jnp.int32(n_u))

        def edge_body(i, _):
            g = c * nr_half + s * (n_chunks * KJ) + i * KJ
            pltpu.sync_copy(rows_hbm.at[pl.ds(g, KJ)], idx_v)
            _adjust(idx_v, -off_r)
            for j in range(KJ):
                pltpu.sync_copy(
                    ones_v.at[pl.ds(j * 128, 128)],
                    hist_sh.at[idx_v.at[j]],
                    add=True,
                )
            return 0

        lax.fori_loop(0, n_chunks, edge_body, 0)
        plsc.subcore_barrier()

        def hist_chunk(k, _):
            pltpu.sync_copy(hist_sh.at[pl.ds(base_n + k * HCH, HCH)], hbuf)

            def grp(g, _):
                deg = hbuf[pl.ds(g * LANES, LANES)]
                x = deg + 1e-7
                y = _rsqrt16(x)
                b = k * HCH + g * LANES
                di_v[pl.ds(b, LANES)] = y
                iv_v[pl.ds(b, LANES)] = 1.0 / x
                sd_v[pl.ds(b, LANES)] = 0.25 * x * y
                return 0

            lax.fori_loop(0, HCH // LANES, grp, 0)
            return 0

        lax.fori_loop(0, tile_rows // HCH, hist_chunk, 0)
        pltpu.sync_copy(iv_v, ivd_hbm.at[pl.ds(out0, tile_rows)])
        pltpu.sync_copy(sd_v, sqd_hbm.at[pl.ds(out0, tile_rows)])

        def t0_chunk(k, _):
            pltpu.sync_copy(emb_hbm.at[pl.ds(out0 + k * RCH, RCH)], ebuf)
            _scale_rows(ebuf, tbuf, di_v, k * RCH, RCH)
            pltpu.sync_copy(tbuf, t0_hbm.at[pl.ds(out0 + k * RCH, RCH)])
            return 0

        lax.fori_loop(0, tile_rows // RCH, t0_chunk, 0)

    return pre


def _make_layer(e_pad, pad_half, n_u, tile_rows):
    tchunks = e_pad // K
    n_chunks = tchunks // NS
    mesh = _mesh()
    SL = 3
    J2 = 2 * KJ

    @functools.partial(
        pl.kernel,
        mesh=mesh,
        compiler_params=pltpu.CompilerParams(use_tc_tiling_on_sc=False),
        out_type=jax.ShapeDtypeStruct((NC * pad_half, 32), jnp.float32),
        scratch_types=[
            pltpu.VMEM_SHARED((pad_half, 32), jnp.float32),
            pltpu.VMEM((SL * K, 32), jnp.float32),
            pltpu.VMEM((SL * J2, 128), jnp.int32),
            pltpu.VMEM((LRCH,), jnp.float32),
            pltpu.SemaphoreType.DMA,
            pltpu.SemaphoreType.DMA,
            pltpu.SemaphoreType.DMA,
        ],
    )
    def layer(comb_hbm, t_hbm, ivd_hbm, out_hbm,
              acc_sh, gbuf, idx_v, ivc, sem_g, sem_s, sem_i):
        c = lax.axis_index("c")
        s = lax.axis_index("s")
        base_n = s * tile_rows
        out0 = c * pad_half + base_n

        zv = jnp.zeros((LANES,), jnp.float32)

        def zrow(r, _):
            gbuf[r, pl.ds(0, 16)] = zv
            gbuf[r, pl.ds(16, 16)] = zv
            return 0

        lax.fori_loop(0, LRCH, zrow, 0)
        for k in range(tile_rows // LRCH):
            pltpu.sync_copy(
                gbuf.at[pl.ds(0, LRCH)],
                acc_sh.at[pl.ds(base_n + k * LRCH, LRCH)],
            )
        plsc.subcore_barrier()

        off_r = jnp.where(c == 0, jnp.int32(0), jnp.int32(n_u))
        off_c = jnp.where(c == 0, jnp.int32(pad_half - n_u), jnp.int32(0))
        gbase = (c * tchunks + s * n_chunks) * J2

        def idx_copy(i, slot):
            return pltpu.make_async_copy(
                comb_hbm.at[pl.ds(gbase + i * J2, J2)],
                idx_v.at[pl.ds(slot * J2, J2)],
                sem_i,
            )

        def adjust(slot):
            for j in range(KJ):
                def adj(k, _):
                    idx_v[slot * J2 + j, pl.ds(k * LANES, LANES)] = (
                        idx_v[slot * J2 + j, pl.ds(k * LANES, LANES)] - off_r
                    )
                    idx_v[slot * J2 + KJ + j, pl.ds(k * LANES, LANES)] = (
                        idx_v[slot * J2 + KJ + j, pl.ds(k * LANES, LANES)]
                        + off_c
                    )
                    return 0

                lax.fori_loop(0, 128 // LANES, adj, 0)

        def gather(slot):
            return [
                pltpu.make_async_copy(
                    t_hbm.at[idx_v.at[slot * J2 + KJ + j]],
                    gbuf.at[pl.ds(slot * K + j * 128, 128)],
                    sem_g,
                )
                for j in range(KJ)
            ]

        def scatter(slot):
            return [
                pltpu.make_async_copy(
                    gbuf.at[pl.ds(slot * K + j * 128, 128)],
                    acc_sh.at[idx_v.at[slot * J2 + j]],
                    sem_s,
                )
                for j in range(KJ)
            ]

        idx_copy(0, 0).start()
        idx_copy(0, 0).wait()
        adjust(0)
        for d in gather(0):
            d.start()

        @pl.when(n_chunks > 1)
        def _():
            idx_copy(1, 1).start()

        def edge_body(i, _):
            slot = lax.rem(i, SL)
            nxt = lax.rem(i + 1, SL)
            nxt2 = lax.rem(i + 2, SL)

            @pl.when(i >= 1)
            def _():
                for d in scatter(lax.rem(i - 1, SL)):
                    d.wait()

            @pl.when(i + 1 < n_chunks)
            def _():
                idx_copy(i + 1, nxt).wait()
                adjust(nxt)
                for d in gather(nxt):
                    d.start()

            @pl.when(i + 2 < n_chunks)
            def _():
                idx_copy(i + 2, nxt2).start()

            for d in gather(slot):
                d.wait()
            for d in scatter(slot):
                d.start(add=True)
            return 0

        lax.fori_loop(0, n_chunks, edge_body, 0)
        for d in scatter(lax.rem(n_chunks - 1, SL)):
            d.wait()
        plsc.subcore_barrier()

        def drain(k, _):
            pltpu.sync_copy(ivd_hbm.at[pl.ds(out0 + k * LRCH, LRCH)], ivc)
            pltpu.sync_copy(acc_sh.at[pl.ds(base_n + k * LRCH, LRCH)],
                            gbuf.at[pl.ds(0, LRCH)])
            _scale_rows(gbuf, gbuf, ivc, 0, LRCH)
            pltpu.sync_copy(gbuf.at[pl.ds(0, LRCH)],
                            out_hbm.at[pl.ds(out0 + k * LRCH, LRCH)])
            return 0

        lax.fori_loop(0, tile_rows // LRCH, drain, 0)

    return layer


def _make_final(pad_half, n_i, batch, tile_rows):
    upt = batch // NS
    uj = upt // 128
    mesh = _mesh()

    @functools.partial(
        pl.kernel,
        mesh=mesh,
        compiler_params=pltpu.CompilerParams(use_tc_tiling_on_sc=False),
        out_type=(
            jax.ShapeDtypeStruct((batch, 32), jnp.float32),
            jax.ShapeDtypeStruct((n_i, 32), jnp.float32),
        ),
        scratch_types=[
            pltpu.VMEM((RCH,), jnp.float32),
            pltpu.VMEM((RCH, 32), jnp.float32),
            pltpu.VMEM((RCH, 32), jnp.float32),
            pltpu.VMEM((RCH, 32), jnp.float32),
            pltpu.VMEM((RCH, 32), jnp.float32),
            pltpu.VMEM((uj, 128), jnp.int32),
            pltpu.VMEM((RCH, 32), jnp.float32),
            pltpu.SemaphoreType.DMA,
        ],
    )
    def final(uid_hbm, t0_hbm, t1_hbm, t2_hbm, t3_hbm, sqd_hbm,
              uout_hbm, iout_hbm,
              sdb, b0, b1, b2, b3, uid_v, obuf, sem):
        c = lax.axis_index("c")
        s = lax.axis_index("s")
        tts = (t0_hbm, t1_hbm, t2_hbm, t3_hbm)
        bbs = (b0, b1, b2, b3)

        @pl.when(c == 0)
        def _user():
            pltpu.sync_copy(uid_hbm.at[pl.ds(s * uj, uj)], uid_v)
            for j in range(uj):
                pltpu.async_copy(
                    sqd_hbm.at[uid_v.at[j]], sdb.at[pl.ds(j * 128, 128)], sem
                ).wait()
            for t, b in zip(tts, bbs):
                for j in range(uj):
                    pltpu.async_copy(
                        t.at[uid_v.at[j]], b.at[pl.ds(j * 128, 128)], sem
                    ).wait()

            for j in range(uj):
                def grp(g, _):
                    sc16 = sdb[pl.ds(j * 128 + g * LANES, LANES)]
                    for m in range(LANES):
                        r = j * 128 + g * LANES + m
                        sc = sc16[m]
                        obuf[r, pl.ds(0, 16)] = (
                            b0[r, pl.ds(0, 16)] + b1[r, pl.ds(0, 16)]
                            + b2[r, pl.ds(0, 16)] + b3[r, pl.ds(0, 16)]
                        ) * sc
                        obuf[r, pl.ds(16, 16)] = (
                            b0[r, pl.ds(16, 16)] + b1[r, pl.ds(16, 16)]
                            + b2[r, pl.ds(16, 16)] + b3[r, pl.ds(16, 16)]
                        ) * sc
                    return 0

                lax.fori_loop(0, 128 // LANES, grp, 0)
            pltpu.sync_copy(
                obuf.at[pl.ds(0, upt)], uout_hbm.at[pl.ds(s * upt, upt)]
            )

        @pl.when(c == 1)
        def _item():
            def chunk(k, _):
                row0 = jnp.minimum(s * tile_rows + k * RCH, n_i - RCH)
                pltpu.sync_copy(sqd_hbm.at[pl.ds(pad_half + row0, RCH)], sdb)
                for t, b in zip(tts, bbs):
                    pltpu.sync_copy(t.at[pl.ds(pad_half + row0, RCH)], b)

                def grp(g, _):
                    sc16 = sdb[pl.ds(g * LANES, LANES)]
                    for m in range(LANES):
                        r = g * LANES + m
                        sc = sc16[m]
                        obuf[r, pl.ds(0, 16)] = (
                            b0[r, pl.ds(0, 16)] + b1[r, pl.ds(0, 16)]
                            + b2[r, pl.ds(0, 16)] + b3[r, pl.ds(0, 16)]
                        ) * sc
                        obuf[r, pl.ds(16, 16)] = (
                            b0[r, pl.ds(16, 16)] + b1[r, pl.ds(16, 16)]
                            + b2[r, pl.ds(16, 16)] + b3[r, pl.ds(16, 16)]
                        ) * sc
                    return 0

                lax.fori_loop(0, RCH // LANES, grp, 0)
                pltpu.sync_copy(obuf, iout_hbm.at[pl.ds(row0, RCH)])
                return 0

            lax.fori_loop(0, -(-tile_rows // RCH), chunk, 0)

    return final


def kernel(user_id, items, Hs, mask, item_seq, user_price_seq, item_price_seq,
           user_count, nft_count, item_seq_len, user_emb, item_emb,
           adj_rows, adj_cols, adj_vals):
    n_u, d = user_emb.shape
    n_i = item_emb.shape[0]
    e = adj_rows.shape[0] // 2
    batch = user_id.shape[0]

    tile_rows = -(-max(n_u, n_i) // NS)
    tile_rows = -(-tile_rows // RCH) * RCH
    pad_half = NS * tile_rows

    ept = -(-e // (NS * K)) * K
    e_pad = ept * NS
    nr_half = e_pad // 128

    pad_e = e_pad - e
    rows_p = jnp.concatenate([
        jnp.pad(adj_rows[:e], (0, pad_e), constant_values=n_u + 8),
        jnp.pad(adj_rows[e:], (0, pad_e), constant_values=n_u + n_i + 8),
    ]).reshape(2 * nr_half, 128)
    cols_p = jnp.concatenate([
        jnp.pad(adj_cols[:e], (0, pad_e), constant_values=n_u),
        jnp.pad(adj_cols[e:], (0, pad_e), constant_values=0),
    ]).reshape(2 * nr_half, 128)
    tchunks = e_pad // K
    comb = jnp.concatenate([
        rows_p.reshape(2, tchunks, K // 128, 128),
        cols_p.reshape(2, tchunks, K // 128, 128),
    ], axis=2).reshape(2 * tchunks * 2 * (K // 128), 128)
    emb_pad = jnp.concatenate([
        user_emb,
        jnp.zeros((pad_half - n_u, d), jnp.float32),
        item_emb,
        jnp.zeros((pad_half - n_i, d), jnp.float32),
    ])

    pre = _make_pre(e_pad, pad_half, n_u, n_i, tile_rows)
    layer = _make_layer(e_pad, pad_half, n_u, tile_rows)
    final = _make_final(pad_half, n_i, batch, tile_rows)

    t0, invdeg, sqd4 = pre(rows_p, emb_pad)
    t1 = layer(comb, t0, invdeg)
    t2 = layer(comb, t1, invdeg)
    t3 = layer(comb, t2, invdeg)
    uid2d = user_id.reshape(batch // 128, 128)
    user_out, item_all = final(uid2d, t0, t1, t2, t3, sqd4)
    return (user_out, item_all)


# speedup vs baseline: 28.4176x; 1.0605x over previous
import functools

import numpy as np
import jax
import jax.numpy as jnp
from jax import lax
from jax.experimental import pallas as pl
from jax.experimental.pallas import tpu as pltpu
from jax.experimental.pallas import tpu_sc as plsc

NC = 2
NS = 16
LANES = 16
K = 256
KJ = K // 128
RCH = 448
LRCH = 224
HCH = 784
HIST_W = 16

_MAGIC = np.int32(0x5F3759DF)


def _mesh():
    return plsc.VectorSubcoreMesh(
        core_axis_name="c", subcore_axis_name="s", num_cores=NC, num_subcores=NS
    )


def _rsqrt16(x):
    i = lax.bitcast_convert_type(x, jnp.int32)
    y = lax.bitcast_convert_type(
        _MAGIC - lax.shift_right_logical(i, 1), jnp.float32
    )
    for _ in range(3):
        y = y * (1.5 - 0.5 * x * y * y)
    return y


def _fill1d(ref, n, value):
    v = jnp.full((LANES,), value, jnp.float32)

    def body(r, _):
        ref[pl.ds(r * LANES, LANES)] = v
        return 0

    lax.fori_loop(0, n // LANES, body, 0)


def _scale_rows(src, dst, sc_v, base, n_rows):
    def grp(g, _):
        sc16 = sc_v[pl.ds(base + g * LANES, LANES)]
        for m in range(LANES):
            r = g * LANES + m
            sc = sc16[m]
            dst[r, pl.ds(0, 16)] = src[r, pl.ds(0, 16)] * sc
            dst[r, pl.ds(16, 16)] = src[r, pl.ds(16, 16)] * sc
        return 0

    lax.fori_loop(0, n_rows // LANES, grp, 0)


def _adjust(idx_v, off):
    for j in range(KJ):
        def body(k, _):
            idx_v[j, pl.ds(k * LANES, LANES)] = (
                idx_v[j, pl.ds(k * LANES, LANES)] + off
            )
            return 0

        lax.fori_loop(0, 128 // LANES, body, 0)


def _make_pre(e_pad, pad_half, n_u, n_i, tile_rows):
    tchunks = e_pad // K
    n_chunks = tchunks // NS
    mesh = _mesh()
    SL = 3
    J2 = 2 * KJ

    @functools.partial(
        pl.kernel,
        mesh=mesh,
        compiler_params=pltpu.CompilerParams(use_tc_tiling_on_sc=False),
        out_type=(
            jax.ShapeDtypeStruct((NC * pad_half, 32), jnp.float32),
            jax.ShapeDtypeStruct((NC * pad_half,), jnp.float32),
            jax.ShapeDtypeStruct((NC * pad_half,), jnp.float32),
        ),
        scratch_types=[
            pltpu.VMEM_SHARED((pad_half,), jnp.float32),
            pltpu.VMEM((K,), jnp.float32),
            pltpu.VMEM((SL * KJ, 128), jnp.int32),
            pltpu.VMEM((HCH,), jnp.float32),
            pltpu.VMEM((tile_rows,), jnp.float32),
            pltpu.VMEM((tile_rows,), jnp.float32),
            pltpu.VMEM((tile_rows,), jnp.float32),
            pltpu.VMEM((2 * RCH, 32), jnp.float32),
            pltpu.SemaphoreType.DMA,
            pltpu.SemaphoreType.DMA,
            pltpu.SemaphoreType.DMA,
            pltpu.SemaphoreType.DMA,
        ],
    )
    def pre(comb_hbm, emb_hbm, t0_hbm, ivd_hbm, sqd_hbm,
            hist_sh, ones_v, idx_v, hbuf, di_v, iv_v, sd_v, ebuf,
            sem_i, sem_s, sem_e, sem_o):
        c = lax.axis_index("c")
        s = lax.axis_index("s")
        base_n = s * tile_rows
        out0 = c * pad_half + base_n

        _fill1d(ones_v, K, 1.0)
        _fill1d(hbuf, HCH, 0.0)
        for k in range(tile_rows // HCH):
            pltpu.sync_copy(hbuf, hist_sh.at[pl.ds(base_n + k * HCH, HCH)])
        plsc.subcore_barrier()

        off_r = jnp.where(c == 0, jnp.int32(0), jnp.int32(n_u))
        gbase = (c * tchunks + s * n_chunks) * J2

        def idx_copy(i, slot):
            return pltpu.make_async_copy(
                comb_hbm.at[pl.ds(gbase + i * J2, KJ)],
                idx_v.at[pl.ds(slot * KJ, KJ)],
                sem_i,
            )

        def adjust(slot):
            for j in range(KJ):
                def adj(k, _):
                    idx_v[slot * KJ + j, pl.ds(k * LANES, LANES)] = (
                        idx_v[slot * KJ + j, pl.ds(k * LANES, LANES)] - off_r
                    )
                    return 0

                lax.fori_loop(0, 128 // LANES, adj, 0)

        def scatter(slot):
            return [
                pltpu.make_async_copy(
                    ones_v.at[pl.ds(j * 128, 128)],
                    hist_sh.at[idx_v.at[slot * KJ + j]],
                    sem_s,
                )
                for j in range(KJ)
            ]

        idx_copy(0, 0).start()
        idx_copy(0, 0).wait()
        adjust(0)

        @pl.when(n_chunks > 1)
        def _():
            idx_copy(1, 1).start()

        def edge_body(i, _):
            slot = lax.rem(i, SL)
            nxt = lax.rem(i + 1, SL)
            nxt2 = lax.rem(i + 2, SL)

            @pl.when(i >= 1)
            def _():
                for d in scatter(lax.rem(i - 1, SL)):
                    d.wait()

            @pl.when(i + 1 < n_chunks)
            def _():
                idx_copy(i + 1, nxt).wait()
                adjust(nxt)

            @pl.when(i + 2 < n_chunks)
            def _():
                idx_copy(i + 2, nxt2).start()

            for d in scatter(slot):
                d.start(add=True)
            return 0

        lax.fori_loop(0, n_chunks, edge_body, 0)
        for d in scatter(lax.rem(n_chunks - 1, SL)):
            d.wait()
        plsc.subcore_barrier()

        def hist_chunk(k, _):
            pltpu.sync_copy(hist_sh.at[pl.ds(base_n + k * HCH, HCH)], hbuf)

            def grp(g, _):
                deg = hbuf[pl.ds(g * LANES, LANES)]
                x = deg + 1e-7
                y = _rsqrt16(x)
                b = k * HCH + g * LANES
                di_v[pl.ds(b, LANES)] = y
                iv_v[pl.ds(b, LANES)] = 1.0 / x
                sd_v[pl.ds(b, LANES)] = 0.25 * x * y
                return 0

            lax.fori_loop(0, HCH // LANES, grp, 0)
            return 0

        lax.fori_loop(0, tile_rows // HCH, hist_chunk, 0)
        pltpu.sync_copy(iv_v, ivd_hbm.at[pl.ds(out0, tile_rows)])
        pltpu.sync_copy(sd_v, sqd_hbm.at[pl.ds(out0, tile_rows)])

        n_t0 = tile_rows // RCH

        def eload(k, slot):
            return pltpu.make_async_copy(
                emb_hbm.at[pl.ds(out0 + k * RCH, RCH)],
                ebuf.at[pl.ds(slot * RCH, RCH)],
                sem_e,
            )

        def estore(k, slot):
            return pltpu.make_async_copy(
                ebuf.at[pl.ds(slot * RCH, RCH)],
                t0_hbm.at[pl.ds(out0 + k * RCH, RCH)],
                sem_o,
            )

        eload(0, 0).start()

        def t0_chunk(k, _):
            slot = lax.rem(k, 2)

            @pl.when(k >= 1)
            def _():
                estore(k - 1, 1 - slot).wait()

            @pl.when(k + 1 < n_t0)
            def _():
                eload(k + 1, 1 - slot).start()

            eload(k, slot).wait()

            def grp(g, _):
                sc16 = di_v[pl.ds(k * RCH + g * LANES, LANES)]
                for m in range(LANES):
                    r = g * LANES + m
                    sc = sc16[m]
                    ro = slot * RCH + r
                    ebuf[ro, pl.ds(0, 16)] = ebuf[ro, pl.ds(0, 16)] * sc
                    ebuf[ro, pl.ds(16, 16)] = ebuf[ro, pl.ds(16, 16)] * sc
                return 0

            lax.fori_loop(0, RCH // LANES, grp, 0)
            estore(k, slot).start()
            return 0

        lax.fori_loop(0, n_t0, t0_chunk, 0)
        estore(n_t0 - 1, jnp.int32((n_t0 - 1) % 2)).wait()

    return pre


def _make_layer(e_pad, pad_half, n_u, tile_rows):
    tchunks = e_pad // K
    n_chunks = tchunks // NS
    mesh = _mesh()
    SL = 3
    J2 = 2 * KJ

    @functools.partial(
        pl.kernel,
        mesh=mesh,
        compiler_params=pltpu.CompilerParams(use_tc_tiling_on_sc=False),
        out_type=jax.ShapeDtypeStruct((NC * pad_half, 32), jnp.float32),
        scratch_types=[
            pltpu.VMEM_SHARED((pad_half, 32), jnp.float32),
            pltpu.VMEM((SL * K, 32), jnp.float32),
            pltpu.VMEM((SL * J2, 128), jnp.int32),
            pltpu.VMEM((LRCH,), jnp.float32),
            pltpu.SemaphoreType.DMA,
            pltpu.SemaphoreType.DMA,
            pltpu.SemaphoreType.DMA,
        ],
    )
    def layer(comb_hbm, t_hbm, ivd_hbm, out_hbm,
              acc_sh, gbuf, idx_v, ivc, sem_g, sem_s, sem_i):
        c = lax.axis_index("c")
        s = lax.axis_index("s")
        base_n = s * tile_rows
        out0 = c * pad_half + base_n

        zv = jnp.zeros((LANES,), jnp.float32)

        def zrow(r, _):
            gbuf[r, pl.ds(0, 16)] = zv
            gbuf[r, pl.ds(16, 16)] = zv
            return 0

        lax.fori_loop(0, LRCH, zrow, 0)
        for k in range(tile_rows // LRCH):
            pltpu.sync_copy(
                gbuf.at[pl.ds(0, LRCH)],
                acc_sh.at[pl.ds(base_n + k * LRCH, LRCH)],
            )
        plsc.subcore_barrier()

        off_r = jnp.where(c == 0, jnp.int32(0), jnp.int32(n_u))
        off_c = jnp.where(c == 0, jnp.int32(pad_half - n_u), jnp.int32(0))
        gbase = (c * tchunks + s * n_chunks) * J2

        def idx_copy(i, slot):
            return pltpu.make_async_copy(
                comb_hbm.at[pl.ds(gbase + i * J2, J2)],
                idx_v.at[pl.ds(slot * J2, J2)],
                sem_i,
            )

        def adjust(slot):
            for j in range(KJ):
                def adj(k, _):
                    idx_v[slot * J2 + j, pl.ds(k * LANES, LANES)] = (
                        idx_v[slot * J2 + j, pl.ds(k * LANES, LANES)] - off_r
                    )
                    idx_v[slot * J2 + KJ + j, pl.ds(k * LANES, LANES)] = (
                        idx_v[slot * J2 + KJ + j, pl.ds(k * LANES, LANES)]
                        + off_c
                    )
                    return 0

                lax.fori_loop(0, 128 // LANES, adj, 0)

        def gather(slot):
            return [
                pltpu.make_async_copy(
                    t_hbm.at[idx_v.at[slot * J2 + KJ + j]],
                    gbuf.at[pl.ds(slot * K + j * 128, 128)],
                    sem_g,
                )
                for j in range(KJ)
            ]

        def scatter(slot):
            return [
                pltpu.make_async_copy(
                    gbuf.at[pl.ds(slot * K + j * 128, 128)],
                    acc_sh.at[idx_v.at[slot * J2 + j]],
                    sem_s,
                )
                for j in range(KJ)
            ]

        idx_copy(0, 0).start()
        idx_copy(0, 0).wait()
        adjust(0)
        for d in gather(0):
            d.start()

        @pl.when(n_chunks > 1)
        def _():
            idx_copy(1, 1).start()

        def edge_body(i, _):
            slot = lax.rem(i, SL)
            nxt = lax.rem(i + 1, SL)
            nxt2 = lax.rem(i + 2, SL)

            @pl.when(i >= 1)
            def _():
                for d in scatter(lax.rem(i - 1, SL)):
                    d.wait()

            @pl.when(i + 1 < n_chunks)
            def _():
                idx_copy(i + 1, nxt).wait()
                adjust(nxt)
                for d in gather(nxt):
                    d.start()

            @pl.when(i + 2 < n_chunks)
            def _():
                idx_copy(i + 2, nxt2).start()

            for d in gather(slot):
                d.wait()
            for d in scatter(slot):
                d.start(add=True)
            return 0

        lax.fori_loop(0, n_chunks, edge_body, 0)
        for d in scatter(lax.rem(n_chunks - 1, SL)):
            d.wait()
        plsc.subcore_barrier()

        def drain(k, _):
            pltpu.sync_copy(ivd_hbm.at[pl.ds(out0 + k * LRCH, LRCH)], ivc)
            pltpu.sync_copy(acc_sh.at[pl.ds(base_n + k * LRCH, LRCH)],
                            gbuf.at[pl.ds(0, LRCH)])
            _scale_rows(gbuf, gbuf, ivc, 0, LRCH)
            pltpu.sync_copy(gbuf.at[pl.ds(0, LRCH)],
                            out_hbm.at[pl.ds(out0 + k * LRCH, LRCH)])
            return 0

        lax.fori_loop(0, tile_rows // LRCH, drain, 0)

    return layer


def _make_final(pad_half, n_i, batch, tile_rows):
    upt = batch // NS
    uj = upt // 128
    mesh = _mesh()

    @functools.partial(
        pl.kernel,
        mesh=mesh,
        compiler_params=pltpu.CompilerParams(use_tc_tiling_on_sc=False),
        out_type=(
            jax.ShapeDtypeStruct((batch, 32), jnp.float32),
            jax.ShapeDtypeStruct((n_i, 32), jnp.float32),
        ),
        scratch_types=[
            pltpu.VMEM((RCH,), jnp.float32),
            pltpu.VMEM((RCH, 32), jnp.float32),
            pltpu.VMEM((RCH, 32), jnp.float32),
            pltpu.VMEM((RCH, 32), jnp.float32),
            pltpu.VMEM((RCH, 32), jnp.float32),
            pltpu.VMEM((uj, 128), jnp.int32),
            pltpu.VMEM((RCH, 32), jnp.float32),
            pltpu.SemaphoreType.DMA,
        ],
    )
    def final(uid_hbm, t0_hbm, t1_hbm, t2_hbm, t3_hbm, sqd_hbm,
              uout_hbm, iout_hbm,
              sdb, b0, b1, b2, b3, uid_v, obuf, sem):
        c = lax.axis_index("c")
        s = lax.axis_index("s")
        tts = (t0_hbm, t1_hbm, t2_hbm, t3_hbm)
        bbs = (b0, b1, b2, b3)

        @pl.when(c == 0)
        def _user():
            pltpu.sync_copy(uid_hbm.at[pl.ds(s * uj, uj)], uid_v)
            for j in range(uj):
                pltpu.async_copy(
                    sqd_hbm.at[uid_v.at[j]], sdb.at[pl.ds(j * 128, 128)], sem
                ).wait()
            for t, b in zip(tts, bbs):
                for j in range(uj):
                    pltpu.async_copy(
                        t.at[uid_v.at[j]], b.at[pl.ds(j * 128, 128)], sem
                    ).wait()

            for j in range(uj):
                def grp(g, _):
                    sc16 = sdb[pl.ds(j * 128 + g * LANES, LANES)]
                    for m in range(LANES):
                        r = j * 128 + g * LANES + m
                        sc = sc16[m]
                        obuf[r, pl.ds(0, 16)] = (
                            b0[r, pl.ds(0, 16)] + b1[r, pl.ds(0, 16)]
                            + b2[r, pl.ds(0, 16)] + b3[r, pl.ds(0, 16)]
                        ) * sc
                        obuf[r, pl.ds(16, 16)] = (
                            b0[r, pl.ds(16, 16)] + b1[r, pl.ds(16, 16)]
                            + b2[r, pl.ds(16, 16)] + b3[r, pl.ds(16, 16)]
                        ) * sc
                    return 0

                lax.fori_loop(0, 128 // LANES, grp, 0)
            pltpu.sync_copy(
                obuf.at[pl.ds(0, upt)], uout_hbm.at[pl.ds(s * upt, upt)]
            )

        @pl.when(c == 1)
        def _item():
            def chunk(k, _):
                row0 = jnp.minimum(s * tile_rows + k * RCH, n_i - RCH)
                pltpu.sync_copy(sqd_hbm.at[pl.ds(pad_half + row0, RCH)], sdb)
                for t, b in zip(tts, bbs):
                    pltpu.sync_copy(t.at[pl.ds(pad_half + row0, RCH)], b)

                def grp(g, _):
                    sc16 = sdb[pl.ds(g * LANES, LANES)]
                    for m in range(LANES):
                        r = g * LANES + m
                        sc = sc16[m]
                        obuf[r, pl.ds(0, 16)] = (
                            b0[r, pl.ds(0, 16)] + b1[r, pl.ds(0, 16)]
                            + b2[r, pl.ds(0, 16)] + b3[r, pl.ds(0, 16)]
                        ) * sc
                        obuf[r, pl.ds(16, 16)] = (
                            b0[r, pl.ds(16, 16)] + b1[r, pl.ds(16, 16)]
                            + b2[r, pl.ds(16, 16)] + b3[r, pl.ds(16, 16)]
                        ) * sc
                    return 0

                lax.fori_loop(0, RCH // LANES, grp, 0)
                pltpu.sync_copy(obuf, iout_hbm.at[pl.ds(row0, RCH)])
                return 0

            lax.fori_loop(0, -(-tile_rows // RCH), chunk, 0)

    return final


def kernel(user_id, items, Hs, mask, item_seq, user_price_seq, item_price_seq,
           user_count, nft_count, item_seq_len, user_emb, item_emb,
           adj_rows, adj_cols, adj_vals):
    n_u, d = user_emb.shape
    n_i = item_emb.shape[0]
    e = adj_rows.shape[0] // 2
    batch = user_id.shape[0]

    tile_rows = -(-max(n_u, n_i) // NS)
    tile_rows = -(-tile_rows // RCH) * RCH
    pad_half = NS * tile_rows

    ept = -(-e // (NS * K)) * K
    e_pad = ept * NS
    nr_half = e_pad // 128

    pad_e = e_pad - e
    rows_p = jnp.concatenate([
        jnp.pad(adj_rows[:e], (0, pad_e), constant_values=n_u + 8),
        jnp.pad(adj_rows[e:], (0, pad_e), constant_values=n_u + n_i + 8),
    ]).reshape(2 * nr_half, 128)
    cols_p = jnp.concatenate([
        jnp.pad(adj_cols[:e], (0, pad_e), constant_values=n_u),
        jnp.pad(adj_cols[e:], (0, pad_e), constant_values=0),
    ]).reshape(2 * nr_half, 128)
    tchunks = e_pad // K
    comb = jnp.concatenate([
        rows_p.reshape(2, tchunks, K // 128, 128),
        cols_p.reshape(2, tchunks, K // 128, 128),
    ], axis=2).reshape(2 * tchunks * 2 * (K // 128), 128)
    emb_pad = jnp.concatenate([
        user_emb,
        jnp.zeros((pad_half - n_u, d), jnp.float32),
        item_emb,
        jnp.zeros((pad_half - n_i, d), jnp.float32),
    ])

    pre = _make_pre(e_pad, pad_half, n_u, n_i, tile_rows)
    layer = _make_layer(e_pad, pad_half, n_u, tile_rows)
    final = _make_final(pad_half, n_i, batch, tile_rows)

    t0, invdeg, sqd4 = pre(comb, emb_pad)
    t1 = layer(comb, t0, invdeg)
    t2 = layer(comb, t1, invdeg)
    t3 = layer(comb, t2, invdeg)
    uid2d = user_id.reshape(batch // 128, 128)
    user_out, item_all = final(uid2d, t0, t1, t2, t3, sqd4)
    return (user_out, item_all)


# speedup vs baseline: 29.2627x; 1.0297x over previous
import functools

import numpy as np
import jax
import jax.numpy as jnp
from jax import lax
from jax.experimental import pallas as pl
from jax.experimental.pallas import tpu as pltpu
from jax.experimental.pallas import tpu_sc as plsc

NC = 2
NS = 16
LANES = 16
K = 256
KJ = K // 128
RCH = 448
LRCH = 224
HCH = 784
HIST_W = 16

_MAGIC = np.int32(0x5F3759DF)


def _mesh():
    return plsc.VectorSubcoreMesh(
        core_axis_name="c", subcore_axis_name="s", num_cores=NC, num_subcores=NS
    )


def _rsqrt16(x):
    i = lax.bitcast_convert_type(x, jnp.int32)
    y = lax.bitcast_convert_type(
        _MAGIC - lax.shift_right_logical(i, 1), jnp.float32
    )
    for _ in range(3):
        y = y * (1.5 - 0.5 * x * y * y)
    return y


def _fill1d(ref, n, value):
    v = jnp.full((LANES,), value, jnp.float32)

    def body(r, _):
        ref[pl.ds(r * LANES, LANES)] = v
        return 0

    lax.fori_loop(0, n // LANES, body, 0)


def _scale_rows(src, dst, sc_v, base, n_rows):
    def grp(g, _):
        sc16 = sc_v[pl.ds(base + g * LANES, LANES)]
        for m in range(LANES):
            r = g * LANES + m
            sc = sc16[m]
            dst[r, pl.ds(0, 16)] = src[r, pl.ds(0, 16)] * sc
            dst[r, pl.ds(16, 16)] = src[r, pl.ds(16, 16)] * sc
        return 0

    lax.fori_loop(0, n_rows // LANES, grp, 0)


def _adjust(idx_v, off):
    for j in range(KJ):
        def body(k, _):
            idx_v[j, pl.ds(k * LANES, LANES)] = (
                idx_v[j, pl.ds(k * LANES, LANES)] + off
            )
            return 0

        lax.fori_loop(0, 128 // LANES, body, 0)


def _make_pre(e_pad, pad_half, n_u, n_i, tile_rows):
    tchunks = e_pad // K
    n_chunks = tchunks // NS
    mesh = _mesh()
    SL = 3
    J2 = 2 * KJ

    @functools.partial(
        pl.kernel,
        mesh=mesh,
        compiler_params=pltpu.CompilerParams(use_tc_tiling_on_sc=False),
        out_type=(
            jax.ShapeDtypeStruct((NC * pad_half, 32), jnp.float32),
            jax.ShapeDtypeStruct((NC * pad_half,), jnp.float32),
            jax.ShapeDtypeStruct((NC * pad_half,), jnp.float32),
        ),
        scratch_types=[
            pltpu.VMEM_SHARED((pad_half,), jnp.float32),
            pltpu.VMEM((K,), jnp.float32),
            pltpu.VMEM((SL * KJ, 128), jnp.int32),
            pltpu.VMEM((HCH,), jnp.float32),
            pltpu.VMEM((tile_rows,), jnp.float32),
            pltpu.VMEM((tile_rows,), jnp.float32),
            pltpu.VMEM((tile_rows,), jnp.float32),
            pltpu.VMEM((2 * RCH, 32), jnp.float32),
            pltpu.SemaphoreType.DMA,
            pltpu.SemaphoreType.DMA,
            pltpu.SemaphoreType.DMA,
            pltpu.SemaphoreType.DMA,
        ],
    )
    def pre(comb_hbm, emb_hbm, t0_hbm, ivd_hbm, sqd_hbm,
            hist_sh, ones_v, idx_v, hbuf, di_v, iv_v, sd_v, ebuf,
            sem_i, sem_s, sem_e, sem_o):
        c = lax.axis_index("c")
        s = lax.axis_index("s")
        base_n = s * tile_rows
        out0 = c * pad_half + base_n

        _fill1d(ones_v, K, 1.0)
        _fill1d(hbuf, HCH, 0.0)
        for k in range(tile_rows // HCH):
            pltpu.sync_copy(hbuf, hist_sh.at[pl.ds(base_n + k * HCH, HCH)])
        plsc.subcore_barrier()

        off_r = jnp.where(c == 0, jnp.int32(0), jnp.int32(n_u))
        gbase = (c * tchunks + s * n_chunks) * J2

        def idx_copy(i, slot):
            return pltpu.make_async_copy(
                comb_hbm.at[pl.ds(gbase + i * J2, KJ)],
                idx_v.at[pl.ds(slot * KJ, KJ)],
                sem_i,
            )

        def adjust(slot):
            for j in range(KJ):
                def adj(k, _):
                    idx_v[slot * KJ + j, pl.ds(k * LANES, LANES)] = (
                        idx_v[slot * KJ + j, pl.ds(k * LANES, LANES)] - off_r
                    )
                    return 0

                lax.fori_loop(0, 128 // LANES, adj, 0)

        def scatter(slot):
            return [
                pltpu.make_async_copy(
                    ones_v.at[pl.ds(j * 128, 128)],
                    hist_sh.at[idx_v.at[slot * KJ + j]],
                    sem_s,
                )
                for j in range(KJ)
            ]

        idx_copy(0, 0).start()
        idx_copy(0, 0).wait()
        adjust(0)

        @pl.when(n_chunks > 1)
        def _():
            idx_copy(1, 1).start()

        def edge_body(i, _):
            slot = lax.rem(i, SL)
            nxt = lax.rem(i + 1, SL)
            nxt2 = lax.rem(i + 2, SL)

            @pl.when(i >= 1)
            def _():
                for d in scatter(lax.rem(i - 1, SL)):
                    d.wait()

            @pl.when(i + 1 < n_chunks)
            def _():
                idx_copy(i + 1, nxt).wait()
                adjust(nxt)

            @pl.when(i + 2 < n_chunks)
            def _():
                idx_copy(i + 2, nxt2).start()

            for d in scatter(slot):
                d.start(add=True)
            return 0

        lax.fori_loop(0, n_chunks, edge_body, 0)
        for d in scatter(lax.rem(n_chunks - 1, SL)):
            d.wait()
        plsc.subcore_barrier()

        def hist_chunk(k, _):
            pltpu.sync_copy(hist_sh.at[pl.ds(base_n + k * HCH, HCH)], hbuf)

            def grp(g, _):
                deg = hbuf[pl.ds(g * LANES, LANES)]
                x = deg + 1e-7
                y = _rsqrt16(x)
                b = k * HCH + g * LANES
                di_v[pl.ds(b, LANES)] = y
                iv_v[pl.ds(b, LANES)] = 1.0 / x
                sd_v[pl.ds(b, LANES)] = 0.25 * x * y
                return 0

            lax.fori_loop(0, HCH // LANES, grp, 0)
            return 0

        lax.fori_loop(0, tile_rows // HCH, hist_chunk, 0)
        pltpu.sync_copy(iv_v, ivd_hbm.at[pl.ds(out0, tile_rows)])
        pltpu.sync_copy(sd_v, sqd_hbm.at[pl.ds(out0, tile_rows)])

        n_t0 = tile_rows // RCH

        def eload(k, slot):
            return pltpu.make_async_copy(
                emb_hbm.at[pl.ds(out0 + k * RCH, RCH)],
                ebuf.at[pl.ds(slot * RCH, RCH)],
                sem_e,
            )

        def estore(k, slot):
            return pltpu.make_async_copy(
                ebuf.at[pl.ds(slot * RCH, RCH)],
                t0_hbm.at[pl.ds(out0 + k * RCH, RCH)],
                sem_o,
            )

        eload(0, 0).start()

        def t0_chunk(k, _):
            slot = lax.rem(k, 2)

            @pl.when(k >= 1)
            def _():
                estore(k - 1, 1 - slot).wait()

            @pl.when(k + 1 < n_t0)
            def _():
                eload(k + 1, 1 - slot).start()

            eload(k, slot).wait()

            def grp(g, _):
                sc16 = di_v[pl.ds(k * RCH + g * LANES, LANES)]
                for m in range(LANES):
                    r = g * LANES + m
                    sc = sc16[m]
                    ro = slot * RCH + r
                    ebuf[ro, pl.ds(0, 16)] = ebuf[ro, pl.ds(0, 16)] * sc
                    ebuf[ro, pl.ds(16, 16)] = ebuf[ro, pl.ds(16, 16)] * sc
                return 0

            lax.fori_loop(0, RCH // LANES, grp, 0)
            estore(k, slot).start()
            return 0

        lax.fori_loop(0, n_t0, t0_chunk, 0)
        estore(n_t0 - 1, jnp.int32((n_t0 - 1) % 2)).wait()

    return pre


def _make_mega(e_pad, pad_half, n_u, tile_rows):
    tchunks = e_pad // K
    n_chunks = tchunks // NS
    mesh = _mesh()
    SL = 3
    J2 = 2 * KJ

    @functools.partial(
        pl.kernel,
        mesh=mesh,
        compiler_params=pltpu.CompilerParams(use_tc_tiling_on_sc=False),
        out_type=(
            jax.ShapeDtypeStruct((NC * pad_half, 32), jnp.float32),
            jax.ShapeDtypeStruct((NC * pad_half, 32), jnp.float32),
            jax.ShapeDtypeStruct((NC * pad_half, 32), jnp.float32),
        ),
        scratch_types=[
            pltpu.VMEM_SHARED((pad_half, 32), jnp.float32),
            pltpu.VMEM((SL * K, 32), jnp.float32),
            pltpu.VMEM((SL * J2, 128), jnp.int32),
            pltpu.VMEM((2 * LRCH,), jnp.float32),
            pltpu.SemaphoreType.DMA,
            pltpu.SemaphoreType.DMA,
            pltpu.SemaphoreType.DMA,
            pltpu.SemaphoreType.DMA,
            pltpu.SemaphoreType.DMA,
        ],
    )
    def mega(comb_hbm, t0_hbm, ivd_hbm, t1_hbm, t2_hbm, t3_hbm,
             acc_sh, gbuf, idx_v, ivc, sem_g, sem_s, sem_i, sem_d, sem_v):
        c = lax.axis_index("c")
        s = lax.axis_index("s")
        base_n = s * tile_rows
        zv = jnp.zeros((LANES,), jnp.float32)
        tts = (t0_hbm, t1_hbm, t2_hbm, t3_hbm)

        for p in range(3):
            src_hbm = tts[p]
            dst_hbm = tts[p + 1]
            dhalf = lax.rem(jnp.int32(p) + c, 2)
            off_r = jnp.where(dhalf == 0, jnp.int32(0), jnp.int32(n_u))
            off_c = jnp.where(dhalf == 0, jnp.int32(pad_half - n_u),
                              jnp.int32(0))
            gbase = (dhalf * tchunks + s * n_chunks) * J2
            out0 = dhalf * pad_half + base_n

            def zrow(r, _):
                gbuf[r, pl.ds(0, 16)] = zv
                gbuf[r, pl.ds(16, 16)] = zv
                return 0

            lax.fori_loop(0, LRCH, zrow, 0)
            for k in range(tile_rows // LRCH):
                pltpu.sync_copy(
                    gbuf.at[pl.ds(0, LRCH)],
                    acc_sh.at[pl.ds(base_n + k * LRCH, LRCH)],
                )
            plsc.subcore_barrier()

            def idx_copy(i, slot, gbase=gbase):
                return pltpu.make_async_copy(
                    comb_hbm.at[pl.ds(gbase + i * J2, J2)],
                    idx_v.at[pl.ds(slot * J2, J2)],
                    sem_i,
                )

            def adjust(slot, off_r=off_r, off_c=off_c):
                for j in range(KJ):
                    def adj(k, _, j=j):
                        idx_v[slot * J2 + j, pl.ds(k * LANES, LANES)] = (
                            idx_v[slot * J2 + j, pl.ds(k * LANES, LANES)]
                            - off_r
                        )
                        idx_v[slot * J2 + KJ + j,
                              pl.ds(k * LANES, LANES)] = (
                            idx_v[slot * J2 + KJ + j,
                                  pl.ds(k * LANES, LANES)] + off_c
                        )
                        return 0

                    lax.fori_loop(0, 128 // LANES, adj, 0)

            def gather(slot, src_hbm=src_hbm):
                return [
                    pltpu.make_async_copy(
                        src_hbm.at[idx_v.at[slot * J2 + KJ + j]],
                        gbuf.at[pl.ds(slot * K + j * 128, 128)],
                        sem_g,
                    )
                    for j in range(KJ)
                ]

            def scatter(slot):
                return [
                    pltpu.make_async_copy(
                        gbuf.at[pl.ds(slot * K + j * 128, 128)],
                        acc_sh.at[idx_v.at[slot * J2 + j]],
                        sem_s,
                    )
                    for j in range(KJ)
                ]

            idx_copy(0, 0).start()
            idx_copy(0, 0).wait()
            adjust(0)
            for d in gather(0):
                d.start()
            idx_copy(1, 1).start()

            def edge_body(i, _, idx_copy=idx_copy, adjust=adjust,
                          gather=gather, scatter=scatter):
                slot = lax.rem(i, SL)
                nxt = lax.rem(i + 1, SL)
                nxt2 = lax.rem(i + 2, SL)

                @pl.when(i >= 1)
                def _():
                    for d in scatter(lax.rem(i - 1, SL)):
                        d.wait()

                @pl.when(i + 1 < n_chunks)
                def _():
                    idx_copy(i + 1, nxt).wait()
                    adjust(nxt)
                    for d in gather(nxt):
                        d.start()

                @pl.when(i + 2 < n_chunks)
                def _():
                    idx_copy(i + 2, nxt2).start()

                for d in gather(slot):
                    d.wait()
                for d in scatter(slot):
                    d.start(add=True)
                return 0

            lax.fori_loop(0, n_chunks, edge_body, 0)
            for d in scatter(lax.rem(n_chunks - 1, SL)):
                d.wait()
            plsc.subcore_barrier()

            n_dr = tile_rows // LRCH

            def iload(k, slot, out0=out0):
                return pltpu.make_async_copy(
                    ivd_hbm.at[pl.ds(out0 + k * LRCH, LRCH)],
                    ivc.at[pl.ds(slot * LRCH, LRCH)],
                    sem_v,
                )

            def dstore(k, slot, dst_hbm=dst_hbm, out0=out0):
                return pltpu.make_async_copy(
                    gbuf.at[pl.ds(slot * LRCH, LRCH)],
                    dst_hbm.at[pl.ds(out0 + k * LRCH, LRCH)],
                    sem_d,
                )

            iload(0, 0).start()

            def drain(k, _, iload=iload, dstore=dstore):
                slot = lax.rem(k, 2)

                @pl.when(k >= 1)
                def _():
                    dstore(k - 1, 1 - slot).wait()

                @pl.when(k + 1 < n_dr)
                def _():
                    iload(k + 1, 1 - slot).start()

                pltpu.sync_copy(acc_sh.at[pl.ds(base_n + k * LRCH, LRCH)],
                                gbuf.at[pl.ds(slot * LRCH, LRCH)])
                iload(k, slot).wait()

                def grp(g, _):
                    sc16 = ivc[pl.ds(slot * LRCH + g * LANES, LANES)]
                    for m in range(LANES):
                        r = slot * LRCH + g * LANES + m
                        sc = sc16[m]
                        gbuf[r, pl.ds(0, 16)] = gbuf[r, pl.ds(0, 16)] * sc
                        gbuf[r, pl.ds(16, 16)] = gbuf[r, pl.ds(16, 16)] * sc
                    return 0

                lax.fori_loop(0, LRCH // LANES, grp, 0)
                dstore(k, slot).start()
                return 0

            lax.fori_loop(0, n_dr, drain, 0)
            dstore(n_dr - 1, jnp.int32((n_dr - 1) % 2)).wait()
            plsc.subcore_barrier()

    return mega


def _make_final(pad_half, n_i, batch, tile_rows):
    upt = batch // NS
    uj = upt // 128
    mesh = _mesh()

    @functools.partial(
        pl.kernel,
        mesh=mesh,
        compiler_params=pltpu.CompilerParams(use_tc_tiling_on_sc=False),
        out_type=(
            jax.ShapeDtypeStruct((batch, 32), jnp.float32),
            jax.ShapeDtypeStruct((n_i, 32), jnp.float32),
        ),
        scratch_types=[
            pltpu.VMEM((RCH,), jnp.float32),
            pltpu.VMEM((RCH, 32), jnp.float32),
            pltpu.VMEM((RCH, 32), jnp.float32),
            pltpu.VMEM((RCH, 32), jnp.float32),
            pltpu.VMEM((RCH, 32), jnp.float32),
            pltpu.VMEM((uj, 128), jnp.int32),
            pltpu.VMEM((RCH, 32), jnp.float32),
            pltpu.SemaphoreType.DMA,
        ],
    )
    def final(uid_hbm, t0_hbm, t1_hbm, t2_hbm, t3_hbm, sqd_hbm,
              uout_hbm, iout_hbm,
              sdb, b0, b1, b2, b3, uid_v, obuf, sem):
        c = lax.axis_index("c")
        s = lax.axis_index("s")
        tts = (t0_hbm, t1_hbm, t2_hbm, t3_hbm)
        bbs = (b0, b1, b2, b3)

        @pl.when(c == 0)
        def _user():
            pltpu.sync_copy(uid_hbm.at[pl.ds(s * uj, uj)], uid_v)
            for j in range(uj):
                pltpu.async_copy(
                    sqd_hbm.at[uid_v.at[j]], sdb.at[pl.ds(j * 128, 128)], sem
                ).wait()
            for t, b in zip(tts, bbs):
                for j in range(uj):
                    pltpu.async_copy(
                        t.at[uid_v.at[j]], b.at[pl.ds(j * 128, 128)], sem
                    ).wait()

            for j in range(uj):
                def grp(g, _):
                    sc16 = sdb[pl.ds(j * 128 + g * LANES, LANES)]
                    for m in range(LANES):
                        r = j * 128 + g * LANES + m
                        sc = sc16[m]
                        obuf[r, pl.ds(0, 16)] = (
                            b0[r, pl.ds(0, 16)] + b1[r, pl.ds(0, 16)]
                            + b2[r, pl.ds(0, 16)] + b3[r, pl.ds(0, 16)]
                        ) * sc
                        obuf[r, pl.ds(16, 16)] = (
                            b0[r, pl.ds(16, 16)] + b1[r, pl.ds(16, 16)]
                            + b2[r, pl.ds(16, 16)] + b3[r, pl.ds(16, 16)]
                        ) * sc
                    return 0

                lax.fori_loop(0, 128 // LANES, grp, 0)
            pltpu.sync_copy(
                obuf.at[pl.ds(0, upt)], uout_hbm.at[pl.ds(s * upt, upt)]
            )

        @pl.when(c == 1)
        def _item():
            def chunk(k, _):
                row0 = jnp.minimum(s * tile_rows + k * RCH, n_i - RCH)
                pltpu.sync_copy(sqd_hbm.at[pl.ds(pad_half + row0, RCH)], sdb)
                for t, b in zip(tts, bbs):
                    pltpu.sync_copy(t.at[pl.ds(pad_half + row0, RCH)], b)

                def grp(g, _):
                    sc16 = sdb[pl.ds(g * LANES, LANES)]
                    for m in range(LANES):
                        r = g * LANES + m
                        sc = sc16[m]
                        obuf[r, pl.ds(0, 16)] = (
                            b0[r, pl.ds(0, 16)] + b1[r, pl.ds(0, 16)]
                            + b2[r, pl.ds(0, 16)] + b3[r, pl.ds(0, 16)]
                        ) * sc
                        obuf[r, pl.ds(16, 16)] = (
                            b0[r, pl.ds(16, 16)] + b1[r, pl.ds(16, 16)]
                            + b2[r, pl.ds(16, 16)] + b3[r, pl.ds(16, 16)]
                        ) * sc
                    return 0

                lax.fori_loop(0, RCH // LANES, grp, 0)
                pltpu.sync_copy(obuf, iout_hbm.at[pl.ds(row0, RCH)])
                return 0

            lax.fori_loop(0, -(-tile_rows // RCH), chunk, 0)

    return final


def kernel(user_id, items, Hs, mask, item_seq, user_price_seq, item_price_seq,
           user_count, nft_count, item_seq_len, user_emb, item_emb,
           adj_rows, adj_cols, adj_vals):
    n_u, d = user_emb.shape
    n_i = item_emb.shape[0]
    e = adj_rows.shape[0] // 2
    batch = user_id.shape[0]

    tile_rows = -(-max(n_u, n_i) // NS)
    tile_rows = -(-tile_rows // RCH) * RCH
    pad_half = NS * tile_rows

    ept = -(-e // (NS * K)) * K
    e_pad = ept * NS
    nr_half = e_pad // 128

    pad_e = e_pad - e
    rows_p = jnp.concatenate([
        jnp.pad(adj_rows[:e], (0, pad_e), constant_values=n_u + 8),
        jnp.pad(adj_rows[e:], (0, pad_e), constant_values=n_u + n_i + 8),
    ]).reshape(2 * nr_half, 128)
    cols_p = jnp.concatenate([
        jnp.pad(adj_cols[:e], (0, pad_e), constant_values=n_u),
        jnp.pad(adj_cols[e:], (0, pad_e), constant_values=0),
    ]).reshape(2 * nr_half, 128)
    tchunks = e_pad // K
    comb = jnp.concatenate([
        rows_p.reshape(2, tchunks, K // 128, 128),
        cols_p.reshape(2, tchunks, K // 128, 128),
    ], axis=2).reshape(2 * tchunks * 2 * (K // 128), 128)
    emb_pad = jnp.concatenate([
        user_emb,
        jnp.zeros((pad_half - n_u, d), jnp.float32),
        item_emb,
        jnp.zeros((pad_half - n_i, d), jnp.float32),
    ])

    pre = _make_pre(e_pad, pad_half, n_u, n_i, tile_rows)
    mega = _make_mega(e_pad, pad_half, n_u, tile_rows)
    final = _make_final(pad_half, n_i, batch, tile_rows)

    t0, invdeg, sqd4 = pre(comb, emb_pad)
    t1, t2, t3 = mega(comb, t0, invdeg)
    uid2d = user_id.reshape(batch // 128, 128)
    user_out, item_all = final(uid2d, t0, t1, t2, t3, sqd4)
    return (user_out, item_all)


# speedup vs baseline: 29.5541x; 1.0100x over previous
import functools

import numpy as np
import jax
import jax.numpy as jnp
from jax import lax
from jax.experimental import pallas as pl
from jax.experimental.pallas import tpu as pltpu
from jax.experimental.pallas import tpu_sc as plsc

NC = 2
NS = 16
LANES = 16
K = 256
KJ = K // 128
RCH = 448
LRCH = 224
HCH = 784
HIST_W = 16

_MAGIC = np.int32(0x5F3759DF)


def _mesh():
    return plsc.VectorSubcoreMesh(
        core_axis_name="c", subcore_axis_name="s", num_cores=NC, num_subcores=NS
    )


def _rsqrt16(x):
    i = lax.bitcast_convert_type(x, jnp.int32)
    y = lax.bitcast_convert_type(
        _MAGIC - lax.shift_right_logical(i, 1), jnp.float32
    )
    for _ in range(3):
        y = y * (1.5 - 0.5 * x * y * y)
    return y


def _fill1d(ref, n, value):
    v = jnp.full((LANES,), value, jnp.float32)

    def body(r, _):
        ref[pl.ds(r * LANES, LANES)] = v
        return 0

    lax.fori_loop(0, n // LANES, body, 0)


def _scale_rows(src, dst, sc_v, base, n_rows):
    def grp(g, _):
        sc16 = sc_v[pl.ds(base + g * LANES, LANES)]
        for m in range(LANES):
            r = g * LANES + m
            sc = sc16[m]
            dst[r, pl.ds(0, 16)] = src[r, pl.ds(0, 16)] * sc
            dst[r, pl.ds(16, 16)] = src[r, pl.ds(16, 16)] * sc
        return 0

    lax.fori_loop(0, n_rows // LANES, grp, 0)


def _adjust(idx_v, off):
    for j in range(KJ):
        def body(k, _):
            idx_v[j, pl.ds(k * LANES, LANES)] = (
                idx_v[j, pl.ds(k * LANES, LANES)] + off
            )
            return 0

        lax.fori_loop(0, 128 // LANES, body, 0)


def _make_pre(e_pad, pad_half, n_u, n_i, tile_rows):
    tchunks = e_pad // K
    n_chunks = tchunks // NS
    mesh = _mesh()
    SL = 3
    J2 = 2 * KJ

    @functools.partial(
        pl.kernel,
        mesh=mesh,
        compiler_params=pltpu.CompilerParams(use_tc_tiling_on_sc=False),
        out_type=(
            jax.ShapeDtypeStruct((NC * pad_half, 32), jnp.float32),
            jax.ShapeDtypeStruct((NC * pad_half,), jnp.float32),
            jax.ShapeDtypeStruct((NC * pad_half,), jnp.float32),
        ),
        scratch_types=[
            pltpu.VMEM_SHARED((pad_half,), jnp.float32),
            pltpu.VMEM((K,), jnp.float32),
            pltpu.VMEM((SL * KJ, 128), jnp.int32),
            pltpu.VMEM((HCH,), jnp.float32),
            pltpu.VMEM((tile_rows,), jnp.float32),
            pltpu.VMEM((tile_rows,), jnp.float32),
            pltpu.VMEM((tile_rows,), jnp.float32),
            pltpu.VMEM((2 * RCH, 32), jnp.float32),
            pltpu.SemaphoreType.DMA,
            pltpu.SemaphoreType.DMA,
            pltpu.SemaphoreType.DMA,
            pltpu.SemaphoreType.DMA,
        ],
    )
    def pre(comb_hbm, emb_hbm, t0_hbm, ivd_hbm, sqd_hbm,
            hist_sh, ones_v, idx_v, hbuf, di_v, iv_v, sd_v, ebuf,
            sem_i, sem_s, sem_e, sem_o):
        c = lax.axis_index("c")
        s = lax.axis_index("s")
        base_n = s * tile_rows
        out0 = c * pad_half + base_n

        _fill1d(ones_v, K, 1.0)
        _fill1d(hbuf, HCH, 0.0)
        for k in range(tile_rows // HCH):
            pltpu.sync_copy(hbuf, hist_sh.at[pl.ds(base_n + k * HCH, HCH)])
        plsc.subcore_barrier()

        off_r = jnp.where(c == 0, jnp.int32(0), jnp.int32(n_u))
        gbase = (c * tchunks + s * n_chunks) * J2

        def idx_copy(i, slot):
            return pltpu.make_async_copy(
                comb_hbm.at[pl.ds(gbase + i * J2, KJ)],
                idx_v.at[pl.ds(slot * KJ, KJ)],
                sem_i,
            )

        def adjust(slot):
            for j in range(KJ):
                def adj(k, _):
                    idx_v[slot * KJ + j, pl.ds(k * LANES, LANES)] = (
                        idx_v[slot * KJ + j, pl.ds(k * LANES, LANES)] - off_r
                    )
                    return 0

                lax.fori_loop(0, 128 // LANES, adj, 0)

        def scatter(slot):
            return [
                pltpu.make_async_copy(
                    ones_v.at[pl.ds(j * 128, 128)],
                    hist_sh.at[idx_v.at[slot * KJ + j]],
                    sem_s,
                )
                for j in range(KJ)
            ]

        idx_copy(0, 0).start()
        idx_copy(0, 0).wait()
        adjust(0)

        @pl.when(n_chunks > 1)
        def _():
            idx_copy(1, 1).start()

        def edge_body(i, _):
            slot = lax.rem(i, SL)
            nxt = lax.rem(i + 1, SL)
            nxt2 = lax.rem(i + 2, SL)

            @pl.when(i >= 1)
            def _():
                for d in scatter(lax.rem(i - 1, SL)):
                    d.wait()

            @pl.when(i + 1 < n_chunks)
            def _():
                idx_copy(i + 1, nxt).wait()
                adjust(nxt)

            @pl.when(i + 2 < n_chunks)
            def _():
                idx_copy(i + 2, nxt2).start()

            for d in scatter(slot):
                d.start(add=True)
            return 0

        lax.fori_loop(0, n_chunks, edge_body, 0)
        for d in scatter(lax.rem(n_chunks - 1, SL)):
            d.wait()
        plsc.subcore_barrier()

        def hist_chunk(k, _):
            pltpu.sync_copy(hist_sh.at[pl.ds(base_n + k * HCH, HCH)], hbuf)

            def grp(g, _):
                deg = hbuf[pl.ds(g * LANES, LANES)]
                x = deg + 1e-7
                y = _rsqrt16(x)
                b = k * HCH + g * LANES
                di_v[pl.ds(b, LANES)] = y
                iv_v[pl.ds(b, LANES)] = 1.0 / x
                sd_v[pl.ds(b, LANES)] = 0.25 * x * y
                return 0

            lax.fori_loop(0, HCH // LANES, grp, 0)
            return 0

        lax.fori_loop(0, tile_rows // HCH, hist_chunk, 0)
        pltpu.sync_copy(iv_v, ivd_hbm.at[pl.ds(out0, tile_rows)])
        pltpu.sync_copy(sd_v, sqd_hbm.at[pl.ds(out0, tile_rows)])

        n_t0 = tile_rows // RCH

        def eload(k, slot):
            return pltpu.make_async_copy(
                emb_hbm.at[pl.ds(out0 + k * RCH, RCH)],
                ebuf.at[pl.ds(slot * RCH, RCH)],
                sem_e,
            )

        def estore(k, slot):
            return pltpu.make_async_copy(
                ebuf.at[pl.ds(slot * RCH, RCH)],
                t0_hbm.at[pl.ds(out0 + k * RCH, RCH)],
                sem_o,
            )

        eload(0, 0).start()

        def t0_chunk(k, _):
            slot = lax.rem(k, 2)

            @pl.when(k >= 1)
            def _():
                estore(k - 1, 1 - slot).wait()

            @pl.when(k + 1 < n_t0)
            def _():
                eload(k + 1, 1 - slot).start()

            eload(k, slot).wait()

            def grp(g, _):
                sc16 = di_v[pl.ds(k * RCH + g * LANES, LANES)]
                for m in range(LANES):
                    r = g * LANES + m
                    sc = sc16[m]
                    ro = slot * RCH + r
                    ebuf[ro, pl.ds(0, 16)] = ebuf[ro, pl.ds(0, 16)] * sc
                    ebuf[ro, pl.ds(16, 16)] = ebuf[ro, pl.ds(16, 16)] * sc
                return 0

            lax.fori_loop(0, RCH // LANES, grp, 0)
            estore(k, slot).start()
            return 0

        lax.fori_loop(0, n_t0, t0_chunk, 0)
        estore(n_t0 - 1, jnp.int32((n_t0 - 1) % 2)).wait()

    return pre


def _make_mega(e_pad, pad_half, n_u, tile_rows):
    tchunks = e_pad // K
    n_chunks = tchunks // NS
    mesh = _mesh()
    SL = 3
    J2 = 2 * KJ

    @functools.partial(
        pl.kernel,
        mesh=mesh,
        compiler_params=pltpu.CompilerParams(use_tc_tiling_on_sc=False),
        out_type=(
            jax.ShapeDtypeStruct((NC * pad_half, 32), jnp.float32),
            jax.ShapeDtypeStruct((NC * pad_half, 32), jnp.float32),
            jax.ShapeDtypeStruct((NC * pad_half, 32), jnp.float32),
        ),
        scratch_types=[
            pltpu.VMEM_SHARED((pad_half, 32), jnp.float32),
            pltpu.VMEM((SL * K, 32), jnp.float32),
            pltpu.VMEM((SL * J2, 128), jnp.int32),
            pltpu.VMEM((2 * LRCH,), jnp.float32),
            pltpu.SemaphoreType.DMA,
            pltpu.SemaphoreType.DMA,
            pltpu.SemaphoreType.DMA,
            pltpu.SemaphoreType.DMA,
            pltpu.SemaphoreType.DMA,
        ],
    )
    def mega(comb_hbm, t0_hbm, ivd_hbm, t1_hbm, t2_hbm, t3_hbm,
             acc_sh, gbuf, idx_v, ivc, sem_g, sem_s, sem_i, sem_d, sem_v):
        c = lax.axis_index("c")
        s = lax.axis_index("s")
        base_n = s * tile_rows
        zv = jnp.zeros((LANES,), jnp.float32)
        tts = (t0_hbm, t1_hbm, t2_hbm, t3_hbm)

        for p in range(3):
            src_hbm = tts[p]
            dst_hbm = tts[p + 1]
            dhalf = lax.rem(jnp.int32(p) + c, 2)
            off_r = jnp.where(dhalf == 0, jnp.int32(0), jnp.int32(n_u))
            off_c = jnp.where(dhalf == 0, jnp.int32(pad_half - n_u),
                              jnp.int32(0))
            gbase = (dhalf * tchunks + s * n_chunks) * J2
            out0 = dhalf * pad_half + base_n

            def zrow(r, _):
                gbuf[r, pl.ds(0, 16)] = zv
                gbuf[r, pl.ds(16, 16)] = zv
                return 0

            lax.fori_loop(0, LRCH, zrow, 0)
            for k in range(tile_rows // LRCH):
                pltpu.sync_copy(
                    gbuf.at[pl.ds(0, LRCH)],
                    acc_sh.at[pl.ds(base_n + k * LRCH, LRCH)],
                )
            plsc.subcore_barrier()

            def idx_copy(i, slot, gbase=gbase):
                return pltpu.make_async_copy(
                    comb_hbm.at[pl.ds(gbase + i * J2, J2)],
                    idx_v.at[pl.ds(slot * J2, J2)],
                    sem_i,
                )

            def adjust(slot, off_r=off_r, off_c=off_c):
                for j in range(KJ):
                    def adj(k, _, j=j):
                        idx_v[slot * J2 + j, pl.ds(k * LANES, LANES)] = (
                            idx_v[slot * J2 + j, pl.ds(k * LANES, LANES)]
                            - off_r
                        )
                        idx_v[slot * J2 + KJ + j,
                              pl.ds(k * LANES, LANES)] = (
                            idx_v[slot * J2 + KJ + j,
                                  pl.ds(k * LANES, LANES)] + off_c
                        )
                        return 0

                    lax.fori_loop(0, 128 // LANES, adj, 0)

            def gather(slot, src_hbm=src_hbm):
                return [
                    pltpu.make_async_copy(
                        src_hbm.at[idx_v.at[slot * J2 + KJ + j]],
                        gbuf.at[pl.ds(slot * K + j * 128, 128)],
                        sem_g,
                    )
                    for j in range(KJ)
                ]

            def scatter(slot):
                return [
                    pltpu.make_async_copy(
                        gbuf.at[pl.ds(slot * K + j * 128, 128)],
                        acc_sh.at[idx_v.at[slot * J2 + j]],
                        sem_s,
                    )
                    for j in range(KJ)
                ]

            idx_copy(0, 0).start()
            idx_copy(0, 0).wait()
            adjust(0)
            for d in gather(0):
                d.start()
            idx_copy(1, 1).start()

            def edge_body(i, _, idx_copy=idx_copy, adjust=adjust,
                          gather=gather, scatter=scatter):
                slot = lax.rem(i, SL)
                nxt = lax.rem(i + 1, SL)
                nxt2 = lax.rem(i + 2, SL)

                @pl.when(i >= 1)
                def _():
                    for d in scatter(lax.rem(i - 1, SL)):
                        d.wait()

                @pl.when(i + 1 < n_chunks)
                def _():
                    idx_copy(i + 1, nxt).wait()
                    adjust(nxt)
                    for d in gather(nxt):
                        d.start()

                @pl.when(i + 2 < n_chunks)
                def _():
                    idx_copy(i + 2, nxt2).start()

                for d in gather(slot):
                    d.wait()
                for d in scatter(slot):
                    d.start(add=True)
                return 0

            lax.fori_loop(0, n_chunks, edge_body, 0)
            for d in scatter(lax.rem(n_chunks - 1, SL)):
                d.wait()
            plsc.subcore_barrier()

            n_dr = tile_rows // LRCH

            def iload(k, slot, out0=out0):
                return pltpu.make_async_copy(
                    ivd_hbm.at[pl.ds(out0 + k * LRCH, LRCH)],
                    ivc.at[pl.ds(slot * LRCH, LRCH)],
                    sem_v,
                )

            def dstore(k, slot, dst_hbm=dst_hbm, out0=out0):
                return pltpu.make_async_copy(
                    gbuf.at[pl.ds(slot * LRCH, LRCH)],
                    dst_hbm.at[pl.ds(out0 + k * LRCH, LRCH)],
                    sem_d,
                )

            iload(0, 0).start()

            def drain(k, _, iload=iload, dstore=dstore):
                slot = lax.rem(k, 2)

                @pl.when(k >= 1)
                def _():
                    dstore(k - 1, 1 - slot).wait()

                @pl.when(k + 1 < n_dr)
                def _():
                    iload(k + 1, 1 - slot).start()

                pltpu.sync_copy(acc_sh.at[pl.ds(base_n + k * LRCH, LRCH)],
                                gbuf.at[pl.ds(slot * LRCH, LRCH)])
                iload(k, slot).wait()

                def grp(g, _):
                    sc16 = ivc[pl.ds(slot * LRCH + g * LANES, LANES)]
                    for m in range(LANES):
                        r = slot * LRCH + g * LANES + m
                        sc = sc16[m]
                        gbuf[r, pl.ds(0, 16)] = gbuf[r, pl.ds(0, 16)] * sc
                        gbuf[r, pl.ds(16, 16)] = gbuf[r, pl.ds(16, 16)] * sc
                    return 0

                lax.fori_loop(0, LRCH // LANES, grp, 0)
                dstore(k, slot).start()
                return 0

            lax.fori_loop(0, n_dr, drain, 0)
            dstore(n_dr - 1, jnp.int32((n_dr - 1) % 2)).wait()
            plsc.subcore_barrier()

    return mega


def _make_final(pad_half, n_i, batch, tile_rows):
    upt = batch // NS
    uj = upt // 128
    mesh = _mesh()

    @functools.partial(
        pl.kernel,
        mesh=mesh,
        compiler_params=pltpu.CompilerParams(use_tc_tiling_on_sc=False),
        out_type=(
            jax.ShapeDtypeStruct((batch, 32), jnp.float32),
            jax.ShapeDtypeStruct((n_i, 32), jnp.float32),
        ),
        scratch_types=[
            pltpu.VMEM((RCH,), jnp.float32),
            pltpu.VMEM((RCH, 32), jnp.float32),
            pltpu.VMEM((RCH, 32), jnp.float32),
            pltpu.VMEM((RCH, 32), jnp.float32),
            pltpu.VMEM((RCH, 32), jnp.float32),
            pltpu.VMEM((uj, 128), jnp.int32),
            pltpu.VMEM((RCH, 32), jnp.float32),
            pltpu.SemaphoreType.DMA,
        ],
    )
    def final(uid_hbm, t0_hbm, t1_hbm, t2_hbm, t3_hbm, sqd_hbm,
              uout_hbm, iout_hbm,
              sdb, b0, b1, b2, b3, uid_v, obuf, sem):
        c = lax.axis_index("c")
        s = lax.axis_index("s")
        tts = (t0_hbm, t1_hbm, t2_hbm, t3_hbm)
        bbs = (b0, b1, b2, b3)

        @pl.when(c == 0)
        def _user():
            pltpu.sync_copy(uid_hbm.at[pl.ds(s * uj, uj)], uid_v)
            for j in range(uj):
                pltpu.async_copy(
                    sqd_hbm.at[uid_v.at[j]], sdb.at[pl.ds(j * 128, 128)], sem
                ).wait()
            for t, b in zip(tts, bbs):
                for j in range(uj):
                    pltpu.async_copy(
                        t.at[uid_v.at[j]], b.at[pl.ds(j * 128, 128)], sem
                    ).wait()

            for j in range(uj):
                def grp(g, _):
                    sc16 = sdb[pl.ds(j * 128 + g * LANES, LANES)]
                    for m in range(LANES):
                        r = j * 128 + g * LANES + m
                        sc = sc16[m]
                        obuf[r, pl.ds(0, 16)] = (
                            b0[r, pl.ds(0, 16)] + b1[r, pl.ds(0, 16)]
                            + b2[r, pl.ds(0, 16)] + b3[r, pl.ds(0, 16)]
                        ) * sc
                        obuf[r, pl.ds(16, 16)] = (
                            b0[r, pl.ds(16, 16)] + b1[r, pl.ds(16, 16)]
                            + b2[r, pl.ds(16, 16)] + b3[r, pl.ds(16, 16)]
                        ) * sc
                    return 0

                lax.fori_loop(0, 128 // LANES, grp, 0)
            pltpu.sync_copy(
                obuf.at[pl.ds(0, upt)], uout_hbm.at[pl.ds(s * upt, upt)]
            )

        half_i = n_i // 2
        tspan = -(-(half_i // NS) // LANES) * LANES
        n_ich = -(-tspan // RCH)

        def chunk(k, _):
            row0 = jnp.minimum(c * half_i + s * tspan + k * RCH,
                               n_i - RCH)
            pltpu.sync_copy(sqd_hbm.at[pl.ds(pad_half + row0, RCH)], sdb)
            for t, b in zip(tts, bbs):
                pltpu.sync_copy(
                    t.at[pl.ds(pad_half + row0, RCH)], b
                )

            def grp(g, _):
                sc16 = sdb[pl.ds(g * LANES, LANES)]
                for m in range(LANES):
                    r = g * LANES + m
                    sc = sc16[m]
                    obuf[r, pl.ds(0, 16)] = (
                        b0[r, pl.ds(0, 16)] + b1[r, pl.ds(0, 16)]
                        + b2[r, pl.ds(0, 16)] + b3[r, pl.ds(0, 16)]
                    ) * sc
                    obuf[r, pl.ds(16, 16)] = (
                        b0[r, pl.ds(16, 16)] + b1[r, pl.ds(16, 16)]
                        + b2[r, pl.ds(16, 16)] + b3[r, pl.ds(16, 16)]
                    ) * sc
                return 0

            lax.fori_loop(0, RCH // LANES, grp, 0)
            pltpu.sync_copy(obuf, iout_hbm.at[pl.ds(row0, RCH)])
            return 0

        lax.fori_loop(0, n_ich, chunk, 0)

    return final


def kernel(user_id, items, Hs, mask, item_seq, user_price_seq, item_price_seq,
           user_count, nft_count, item_seq_len, user_emb, item_emb,
           adj_rows, adj_cols, adj_vals):
    n_u, d = user_emb.shape
    n_i = item_emb.shape[0]
    e = adj_rows.shape[0] // 2
    batch = user_id.shape[0]

    tile_rows = -(-max(n_u, n_i) // NS)
    tile_rows = -(-tile_rows // RCH) * RCH
    pad_half = NS * tile_rows

    ept = -(-e // (NS * K)) * K
    e_pad = ept * NS
    nr_half = e_pad // 128

    pad_e = e_pad - e
    rows_p = jnp.concatenate([
        jnp.pad(adj_rows[:e], (0, pad_e), constant_values=n_u + 8),
        jnp.pad(adj_rows[e:], (0, pad_e), constant_values=n_u + n_i + 8),
    ]).reshape(2 * nr_half, 128)
    cols_p = jnp.concatenate([
        jnp.pad(adj_cols[:e], (0, pad_e), constant_values=n_u),
        jnp.pad(adj_cols[e:], (0, pad_e), constant_values=0),
    ]).reshape(2 * nr_half, 128)
    tchunks = e_pad // K
    comb = jnp.concatenate([
        rows_p.reshape(2, tchunks, K // 128, 128),
        cols_p.reshape(2, tchunks, K // 128, 128),
    ], axis=2).reshape(2 * tchunks * 2 * (K // 128), 128)
    emb_pad = jnp.concatenate([
        user_emb,
        jnp.zeros((pad_half - n_u, d), jnp.float32),
        item_emb,
        jnp.zeros((pad_half - n_i, d), jnp.float32),
    ])

    pre = _make_pre(e_pad, pad_half, n_u, n_i, tile_rows)
    mega = _make_mega(e_pad, pad_half, n_u, tile_rows)
    final = _make_final(pad_half, n_i, batch, tile_rows)

    t0, invdeg, sqd4 = pre(comb, emb_pad)
    t1, t2, t3 = mega(comb, t0, invdeg)
    uid2d = user_id.reshape(batch // 128, 128)
    user_out, item_all = final(uid2d, t0, t1, t2, t3, sqd4)
    return (user_out, item_all)


# speedup vs baseline: 33.6137x; 1.1374x over previous
import functools

import numpy as np
import jax
import jax.numpy as jnp
from jax import lax
from jax.experimental import pallas as pl
from jax.experimental.pallas import tpu as pltpu
from jax.experimental.pallas import tpu_sc as plsc

NC = 2
NS = 16
LANES = 16
K = 256
KJ = K // 128
RCH = 448
LRCH = 224
HCH = 784
HIST_W = 16

_MAGIC = np.int32(0x5F3759DF)


def _mesh():
    return plsc.VectorSubcoreMesh(
        core_axis_name="c", subcore_axis_name="s", num_cores=NC, num_subcores=NS
    )


def _rsqrt16(x):
    i = lax.bitcast_convert_type(x, jnp.int32)
    y = lax.bitcast_convert_type(
        _MAGIC - lax.shift_right_logical(i, 1), jnp.float32
    )
    for _ in range(3):
        y = y * (1.5 - 0.5 * x * y * y)
    return y


def _fill1d(ref, n, value):
    v = jnp.full((LANES,), value, jnp.float32)

    def body(r, _):
        ref[pl.ds(r * LANES, LANES)] = v
        return 0

    lax.fori_loop(0, n // LANES, body, 0)


def _scale_rows(src, dst, sc_v, base, n_rows):
    def grp(g, _):
        sc16 = sc_v[pl.ds(base + g * LANES, LANES)]
        for m in range(LANES):
            r = g * LANES + m
            sc = sc16[m]
            dst[r, pl.ds(0, 16)] = src[r, pl.ds(0, 16)] * sc
            dst[r, pl.ds(16, 16)] = src[r, pl.ds(16, 16)] * sc
        return 0

    lax.fori_loop(0, n_rows // LANES, grp, 0)


def _adjust(idx_v, off):
    for j in range(KJ):
        def body(k, _):
            idx_v[j, pl.ds(k * LANES, LANES)] = (
                idx_v[j, pl.ds(k * LANES, LANES)] + off
            )
            return 0

        lax.fori_loop(0, 128 // LANES, body, 0)


def _make_pre(e_pad, pad_half, n_u, n_i, tile_rows):
    nr_half = e_pad // 128
    epr = nr_half // NS
    KP = 6
    n_chunks = epr // KP
    mesh = _mesh()
    SL = 3

    @functools.partial(
        pl.kernel,
        mesh=mesh,
        compiler_params=pltpu.CompilerParams(use_tc_tiling_on_sc=False),
        out_type=(
            jax.ShapeDtypeStruct((NC * pad_half, 32), jnp.float32),
            jax.ShapeDtypeStruct((NC * pad_half,), jnp.float32),
            jax.ShapeDtypeStruct((NC * pad_half,), jnp.float32),
        ),
        scratch_types=[
            pltpu.VMEM_SHARED((pad_half,), jnp.float32),
            pltpu.VMEM((KP * 128,), jnp.float32),
            pltpu.VMEM((SL * KP, 128), jnp.int32),
            pltpu.VMEM((HCH,), jnp.float32),
            pltpu.VMEM((tile_rows,), jnp.float32),
            pltpu.VMEM((tile_rows,), jnp.float32),
            pltpu.VMEM((tile_rows,), jnp.float32),
            pltpu.VMEM((2 * RCH, 32), jnp.float32),
            pltpu.SemaphoreType.DMA,
            pltpu.SemaphoreType.DMA,
            pltpu.SemaphoreType.DMA,
            pltpu.SemaphoreType.DMA,
        ],
    )
    def pre(rows_hbm, uemb_hbm, iemb_hbm, t0_hbm, ivd_hbm, sqd_hbm,
            hist_sh, ones_v, idx_v, hbuf, di_v, iv_v, sd_v, ebuf,
            sem_i, sem_s, sem_e, sem_o):
        c = lax.axis_index("c")
        s = lax.axis_index("s")
        base_n = s * tile_rows
        out0 = c * pad_half + base_n

        _fill1d(ones_v, KP * 128, 1.0)
        _fill1d(hbuf, HCH, 0.0)
        for k in range(tile_rows // HCH):
            pltpu.sync_copy(hbuf, hist_sh.at[pl.ds(base_n + k * HCH, HCH)])
        plsc.subcore_barrier()

        off_r = jnp.where(c == 0, jnp.int32(0), jnp.int32(n_u))
        gbase = c * nr_half + s * epr

        def idx_copy(i, slot):
            return pltpu.make_async_copy(
                rows_hbm.at[pl.ds(gbase + i * KP, KP)],
                idx_v.at[pl.ds(slot * KP, KP)],
                sem_i,
            )

        def adjust(slot):
            for j in range(KP):
                def adj(k, _, j=j):
                    idx_v[slot * KP + j, pl.ds(k * LANES, LANES)] = (
                        idx_v[slot * KP + j, pl.ds(k * LANES, LANES)] - off_r
                    )
                    return 0

                lax.fori_loop(0, 128 // LANES, adj, 0)

        def scatter(slot):
            return [
                pltpu.make_async_copy(
                    ones_v.at[pl.ds(j * 128, 128)],
                    hist_sh.at[idx_v.at[slot * KP + j]],
                    sem_s,
                )
                for j in range(KP)
            ]

        idx_copy(0, 0).start()
        idx_copy(0, 0).wait()
        adjust(0)
        idx_copy(1, 1).start()

        def edge_body(i, _):
            slot = lax.rem(i, SL)
            nxt = lax.rem(i + 1, SL)
            nxt2 = lax.rem(i + 2, SL)

            @pl.when(i >= 1)
            def _():
                for d in scatter(lax.rem(i - 1, SL)):
                    d.wait()

            @pl.when(i + 1 < n_chunks)
            def _():
                idx_copy(i + 1, nxt).wait()
                adjust(nxt)

            @pl.when(i + 2 < n_chunks)
            def _():
                idx_copy(i + 2, nxt2).start()

            for d in scatter(slot):
                d.start(add=True)
            return 0

        lax.fori_loop(0, n_chunks, edge_body, 0)
        for d in scatter(lax.rem(n_chunks - 1, SL)):
            d.wait()
        plsc.subcore_barrier()

        def hist_chunk(k, _):
            pltpu.sync_copy(hist_sh.at[pl.ds(base_n + k * HCH, HCH)], hbuf)

            def grp(g, _):
                deg = hbuf[pl.ds(g * LANES, LANES)]
                x = deg + 1e-7
                y = _rsqrt16(x)
                b = k * HCH + g * LANES
                di_v[pl.ds(b, LANES)] = y
                iv_v[pl.ds(b, LANES)] = 1.0 / x
                sd_v[pl.ds(b, LANES)] = 0.25 * x * y
                return 0

            lax.fori_loop(0, HCH // LANES, grp, 0)
            return 0

        lax.fori_loop(0, tile_rows // HCH, hist_chunk, 0)
        pltpu.sync_copy(iv_v, ivd_hbm.at[pl.ds(out0, tile_rows)])
        pltpu.sync_copy(sd_v, sqd_hbm.at[pl.ds(out0, tile_rows)])

        n_t0 = tile_rows // RCH
        n_half = jnp.where(c == 0, jnp.int32(n_u), jnp.int32(n_i))

        def t0_stage(emb_hbm):
            def eload(k, slot):
                row0 = jnp.minimum(base_n + k * RCH, n_half - RCH)
                return pltpu.make_async_copy(
                    emb_hbm.at[pl.ds(row0, RCH)],
                    ebuf.at[pl.ds(slot * RCH, RCH)],
                    sem_e,
                )

            def estore(k, slot):
                row0 = jnp.minimum(base_n + k * RCH, n_half - RCH)
                return pltpu.make_async_copy(
                    ebuf.at[pl.ds(slot * RCH, RCH)],
                    t0_hbm.at[pl.ds(c * pad_half + row0, RCH)],
                    sem_o,
                )

            eload(0, 0).start()

            def t0_chunk(k, _):
                slot = lax.rem(k, 2)
                loc0 = jnp.minimum(base_n + k * RCH, n_half - RCH) - base_n

                @pl.when(k >= 1)
                def _():
                    estore(k - 1, 1 - slot).wait()

                @pl.when(k + 1 < n_t0)
                def _():
                    eload(k + 1, 1 - slot).start()

                eload(k, slot).wait()

                def grp(g, _):
                    sc16 = di_v[pl.ds(loc0 + g * LANES, LANES)]
                    for m in range(LANES):
                        r = g * LANES + m
                        sc = sc16[m]
                        ro = slot * RCH + r
                        ebuf[ro, pl.ds(0, 16)] = ebuf[ro, pl.ds(0, 16)] * sc
                        ebuf[ro, pl.ds(16, 16)] = (
                            ebuf[ro, pl.ds(16, 16)] * sc
                        )
                    return 0

                lax.fori_loop(0, RCH // LANES, grp, 0)
                estore(k, slot).start()
                return 0

            lax.fori_loop(0, n_t0, t0_chunk, 0)
            estore(n_t0 - 1, jnp.int32((n_t0 - 1) % 2)).wait()

        @pl.when(c == 0)
        def _():
            t0_stage(uemb_hbm)

        @pl.when(c == 1)
        def _():
            t0_stage(iemb_hbm)

    return pre


def _make_mega(e_pad, pad_half, n_u, tile_rows):
    tchunks = e_pad // K
    n_chunks = tchunks // NS
    mesh = _mesh()
    SL = 3
    J2 = 2 * KJ

    @functools.partial(
        pl.kernel,
        mesh=mesh,
        compiler_params=pltpu.CompilerParams(use_tc_tiling_on_sc=False),
        out_type=(
            jax.ShapeDtypeStruct((NC * pad_half, 32), jnp.float32),
            jax.ShapeDtypeStruct((NC * pad_half, 32), jnp.float32),
            jax.ShapeDtypeStruct((NC * pad_half, 32), jnp.float32),
        ),
        scratch_types=[
            pltpu.VMEM_SHARED((pad_half, 32), jnp.float32),
            pltpu.VMEM((SL * K, 32), jnp.float32),
            pltpu.VMEM((SL * J2, 128), jnp.int32),
            pltpu.VMEM((2 * LRCH,), jnp.float32),
            pltpu.SemaphoreType.DMA,
            pltpu.SemaphoreType.DMA,
            pltpu.SemaphoreType.DMA,
            pltpu.SemaphoreType.DMA,
            pltpu.SemaphoreType.DMA,
        ],
    )
    def mega(comb_hbm, t0_hbm, ivd_hbm, t1_hbm, t2_hbm, t3_hbm,
             acc_sh, gbuf, idx_v, ivc, sem_g, sem_s, sem_i, sem_d, sem_v):
        c = lax.axis_index("c")
        s = lax.axis_index("s")
        base_n = s * tile_rows
        zv = jnp.zeros((LANES,), jnp.float32)
        tts = (t0_hbm, t1_hbm, t2_hbm, t3_hbm)

        for p in range(3):
            src_hbm = tts[p]
            dst_hbm = tts[p + 1]
            dhalf = lax.rem(jnp.int32(p) + c, 2)
            off_r = jnp.where(dhalf == 0, jnp.int32(0), jnp.int32(n_u))
            off_c = jnp.where(dhalf == 0, jnp.int32(pad_half - n_u),
                              jnp.int32(0))
            gbase = (dhalf * tchunks + s * n_chunks) * J2
            out0 = dhalf * pad_half + base_n

            def zrow(r, _):
                gbuf[r, pl.ds(0, 16)] = zv
                gbuf[r, pl.ds(16, 16)] = zv
                return 0

            lax.fori_loop(0, LRCH, zrow, 0)
            for k in range(tile_rows // LRCH):
                pltpu.sync_copy(
                    gbuf.at[pl.ds(0, LRCH)],
                    acc_sh.at[pl.ds(base_n + k * LRCH, LRCH)],
                )
            plsc.subcore_barrier()

            def idx_copy(i, slot, gbase=gbase):
                return pltpu.make_async_copy(
                    comb_hbm.at[pl.ds(gbase + i * J2, J2)],
                    idx_v.at[pl.ds(slot * J2, J2)],
                    sem_i,
                )

            def adjust(slot, off_r=off_r, off_c=off_c):
                for j in range(KJ):
                    def adj(k, _, j=j):
                        idx_v[slot * J2 + j, pl.ds(k * LANES, LANES)] = (
                            idx_v[slot * J2 + j, pl.ds(k * LANES, LANES)]
                            - off_r
                        )
                        idx_v[slot * J2 + KJ + j,
                              pl.ds(k * LANES, LANES)] = (
                            idx_v[slot * J2 + KJ + j,
                                  pl.ds(k * LANES, LANES)] + off_c
                        )
                        return 0

                    lax.fori_loop(0, 128 // LANES, adj, 0)

            def gather(slot, src_hbm=src_hbm):
                return [
                    pltpu.make_async_copy(
                        src_hbm.at[idx_v.at[slot * J2 + KJ + j]],
                        gbuf.at[pl.ds(slot * K + j * 128, 128)],
                        sem_g,
                    )
                    for j in range(KJ)
                ]

            def scatter(slot):
                return [
                    pltpu.make_async_copy(
                        gbuf.at[pl.ds(slot * K + j * 128, 128)],
                        acc_sh.at[idx_v.at[slot * J2 + j]],
                        sem_s,
                    )
                    for j in range(KJ)
                ]

            idx_copy(0, 0).start()
            idx_copy(0, 0).wait()
            adjust(0)
            for d in gather(0):
                d.start()
            idx_copy(1, 1).start()

            def edge_body(i, _, idx_copy=idx_copy, adjust=adjust,
                          gather=gather, scatter=scatter):
                slot = lax.rem(i, SL)
                nxt = lax.rem(i + 1, SL)
                nxt2 = lax.rem(i + 2, SL)

                @pl.when(i >= 1)
                def _():
                    for d in scatter(lax.rem(i - 1, SL)):
                        d.wait()

                @pl.when(i + 1 < n_chunks)
                def _():
                    idx_copy(i + 1, nxt).wait()
                    adjust(nxt)
                    for d in gather(nxt):
                        d.start()

                @pl.when(i + 2 < n_chunks)
                def _():
                    idx_copy(i + 2, nxt2).start()

                for d in gather(slot):
                    d.wait()
                for d in scatter(slot):
                    d.start(add=True)
                return 0

            lax.fori_loop(0, n_chunks, edge_body, 0)
            for d in scatter(lax.rem(n_chunks - 1, SL)):
                d.wait()
            plsc.subcore_barrier()

            n_dr = tile_rows // LRCH

            def iload(k, slot, out0=out0):
                return pltpu.make_async_copy(
                    ivd_hbm.at[pl.ds(out0 + k * LRCH, LRCH)],
                    ivc.at[pl.ds(slot * LRCH, LRCH)],
                    sem_v,
                )

            def dstore(k, slot, dst_hbm=dst_hbm, out0=out0):
                return pltpu.make_async_copy(
                    gbuf.at[pl.ds(slot * LRCH, LRCH)],
                    dst_hbm.at[pl.ds(out0 + k * LRCH, LRCH)],
                    sem_d,
                )

            iload(0, 0).start()

            def drain(k, _, iload=iload, dstore=dstore):
                slot = lax.rem(k, 2)

                @pl.when(k >= 1)
                def _():
                    dstore(k - 1, 1 - slot).wait()

                @pl.when(k + 1 < n_dr)
                def _():
                    iload(k + 1, 1 - slot).start()

                pltpu.sync_copy(acc_sh.at[pl.ds(base_n + k * LRCH, LRCH)],
                                gbuf.at[pl.ds(slot * LRCH, LRCH)])
                iload(k, slot).wait()

                def grp(g, _):
                    sc16 = ivc[pl.ds(slot * LRCH + g * LANES, LANES)]
                    for m in range(LANES):
                        r = slot * LRCH + g * LANES + m
                        sc = sc16[m]
                        gbuf[r, pl.ds(0, 16)] = gbuf[r, pl.ds(0, 16)] * sc
                        gbuf[r, pl.ds(16, 16)] = gbuf[r, pl.ds(16, 16)] * sc
                    return 0

                lax.fori_loop(0, LRCH // LANES, grp, 0)
                dstore(k, slot).start()
                return 0

            lax.fori_loop(0, n_dr, drain, 0)
            dstore(n_dr - 1, jnp.int32((n_dr - 1) % 2)).wait()
            plsc.subcore_barrier()

    return mega


def _make_final(pad_half, n_i, batch, tile_rows):
    upt = batch // NS
    uj = upt // 128
    mesh = _mesh()

    @functools.partial(
        pl.kernel,
        mesh=mesh,
        compiler_params=pltpu.CompilerParams(use_tc_tiling_on_sc=False),
        out_type=(
            jax.ShapeDtypeStruct((batch, 32), jnp.float32),
            jax.ShapeDtypeStruct((n_i, 32), jnp.float32),
        ),
        scratch_types=[
            pltpu.VMEM((RCH,), jnp.float32),
            pltpu.VMEM((RCH, 32), jnp.float32),
            pltpu.VMEM((RCH, 32), jnp.float32),
            pltpu.VMEM((RCH, 32), jnp.float32),
            pltpu.VMEM((RCH, 32), jnp.float32),
            pltpu.VMEM((uj, 128), jnp.int32),
            pltpu.VMEM((RCH, 32), jnp.float32),
            pltpu.SemaphoreType.DMA,
        ],
    )
    def final(uid_hbm, t0_hbm, t1_hbm, t2_hbm, t3_hbm, sqd_hbm,
              uout_hbm, iout_hbm,
              sdb, b0, b1, b2, b3, uid_v, obuf, sem):
        c = lax.axis_index("c")
        s = lax.axis_index("s")
        tts = (t0_hbm, t1_hbm, t2_hbm, t3_hbm)
        bbs = (b0, b1, b2, b3)

        @pl.when(c == 0)
        def _user():
            pltpu.sync_copy(uid_hbm.at[pl.ds(s * uj, uj)], uid_v)
            for j in range(uj):
                pltpu.async_copy(
                    sqd_hbm.at[uid_v.at[j]], sdb.at[pl.ds(j * 128, 128)], sem
                ).wait()
            for t, b in zip(tts, bbs):
                for j in range(uj):
                    pltpu.async_copy(
                        t.at[uid_v.at[j]], b.at[pl.ds(j * 128, 128)], sem
                    ).wait()

            for j in range(uj):
                def grp(g, _):
                    sc16 = sdb[pl.ds(j * 128 + g * LANES, LANES)]
                    for m in range(LANES):
                        r = j * 128 + g * LANES + m
                        sc = sc16[m]
                        obuf[r, pl.ds(0, 16)] = (
                            b0[r, pl.ds(0, 16)] + b1[r, pl.ds(0, 16)]
                            + b2[r, pl.ds(0, 16)] + b3[r, pl.ds(0, 16)]
                        ) * sc
                        obuf[r, pl.ds(16, 16)] = (
                            b0[r, pl.ds(16, 16)] + b1[r, pl.ds(16, 16)]
                            + b2[r, pl.ds(16, 16)] + b3[r, pl.ds(16, 16)]
                        ) * sc
                    return 0

                lax.fori_loop(0, 128 // LANES, grp, 0)
            pltpu.sync_copy(
                obuf.at[pl.ds(0, upt)], uout_hbm.at[pl.ds(s * upt, upt)]
            )

        half_i = n_i // 2
        tspan = -(-(half_i // NS) // LANES) * LANES
        n_ich = -(-tspan // RCH)

        def chunk(k, _):
            row0 = jnp.minimum(c * half_i + s * tspan + k * RCH,
                               n_i - RCH)
            pltpu.sync_copy(sqd_hbm.at[pl.ds(pad_half + row0, RCH)], sdb)
            for t, b in zip(tts, bbs):
                pltpu.sync_copy(
                    t.at[pl.ds(pad_half + row0, RCH)], b
                )

            def grp(g, _):
                sc16 = sdb[pl.ds(g * LANES, LANES)]
                for m in range(LANES):
                    r = g * LANES + m
                    sc = sc16[m]
                    obuf[r, pl.ds(0, 16)] = (
                        b0[r, pl.ds(0, 16)] + b1[r, pl.ds(0, 16)]
                        + b2[r, pl.ds(0, 16)] + b3[r, pl.ds(0, 16)]
                    ) * sc
                    obuf[r, pl.ds(16, 16)] = (
                        b0[r, pl.ds(16, 16)] + b1[r, pl.ds(16, 16)]
                        + b2[r, pl.ds(16, 16)] + b3[r, pl.ds(16, 16)]
                    ) * sc
                return 0

            lax.fori_loop(0, RCH // LANES, grp, 0)
            pltpu.sync_copy(obuf, iout_hbm.at[pl.ds(row0, RCH)])
            return 0

        lax.fori_loop(0, n_ich, chunk, 0)

    return final


def kernel(user_id, items, Hs, mask, item_seq, user_price_seq, item_price_seq,
           user_count, nft_count, item_seq_len, user_emb, item_emb,
           adj_rows, adj_cols, adj_vals):
    n_u, d = user_emb.shape
    n_i = item_emb.shape[0]
    e = adj_rows.shape[0] // 2
    batch = user_id.shape[0]

    tile_rows = -(-max(n_u, n_i) // NS)
    tile_rows = -(-tile_rows // RCH) * RCH
    pad_half = NS * tile_rows

    ept = -(-e // (NS * K)) * K
    e_pad = ept * NS
    nr_half = e_pad // 128

    pad_e = e_pad - e
    rows_p = jnp.concatenate([
        jnp.pad(adj_rows[:e], (0, pad_e), constant_values=n_u + 8),
        jnp.pad(adj_rows[e:], (0, pad_e), constant_values=n_u + n_i + 8),
    ]).reshape(2 * nr_half, 128)
    cols_p = jnp.concatenate([
        jnp.pad(adj_cols[:e], (0, pad_e), constant_values=n_u),
        jnp.pad(adj_cols[e:], (0, pad_e), constant_values=0),
    ]).reshape(2 * nr_half, 128)
    tchunks = e_pad // K
    comb = jnp.concatenate([
        rows_p.reshape(2, tchunks, K // 128, 128),
        cols_p.reshape(2, tchunks, K // 128, 128),
    ], axis=2).reshape(2 * tchunks * 2 * (K // 128), 128)
    pre = _make_pre(e_pad, pad_half, n_u, n_i, tile_rows)
    mega = _make_mega(e_pad, pad_half, n_u, tile_rows)
    final = _make_final(pad_half, n_i, batch, tile_rows)

    t0, invdeg, sqd4 = pre(rows_p, user_emb, item_emb)
    t1, t2, t3 = mega(comb, t0, invdeg)
    uid2d = user_id.reshape(batch // 128, 128)
    user_out, item_all = final(uid2d, t0, t1, t2, t3, sqd4)
    return (user_out, item_all)
